# baseline (speedup 1.0000x reference)
import functools

import jax
import jax.numpy as jnp
from jax import lax
from jax.experimental import pallas as pl
from jax.experimental.pallas import tpu as pltpu
from jax.experimental.pallas import tpu_sc as plsc

N = 10000
D = 128
H = 64
TF = 64
L = 20
V = 100000
C = 2

NC = 2
NS = 16
NW = NC * NS
CH = 128

NPAD = 10240
ROWS_PER_SUB = NPAD // NS

J_CONV = 82
EP_CONV = NW * J_CONV * CH
CHR = 125
J_RO = 80
NCHUNK_RO = NW * J_RO
CHE = 125
J_EMB = 50
EP_EMB = NW * J_EMB * CHE
assert EP_EMB == N * L


def _mesh():
    return plsc.VectorSubcoreMesh(
        core_axis_name="c", subcore_axis_name="s",
        num_cores=NC, num_subcores=NS)


@functools.cache
def _build_sc_deg_embed():
    return functools.partial(
        pl.kernel,
        out_type=[
            jax.ShapeDtypeStruct((NC, NPAD), jnp.float32),
            jax.ShapeDtypeStruct((EP_EMB, TF), jnp.float32),
        ],
        mesh=_mesh(),
        scratch_types=[
            pltpu.VMEM((J_CONV, CH), jnp.int32),
            pltpu.VMEM((J_EMB, CHE), jnp.int32),
            pltpu.VMEM((CH,), jnp.float32),
            pltpu.VMEM((CHE, TF), jnp.float32),
            pltpu.VMEM((CHE, TF), jnp.float32),
            pltpu.VMEM((ROWS_PER_SUB,), jnp.float32),
            pltpu.VMEM_SHARED((NPAD,), jnp.float32),
            pltpu.SemaphoreType.DMA,
            pltpu.SemaphoreType.DMA,
            pltpu.SemaphoreType.DMA,
            pltpu.SemaphoreType.DMA,
        ],
        compiler_params=pltpu.CompilerParams(use_tc_tiling_on_sc=False),
    )(_sc_deg_embed_body)


def _sc_deg_embed(dst3, xt3, embed):
    return _build_sc_deg_embed()(dst3, xt3, embed)


def _sc_deg_embed_body(dst3_hbm, xt3_hbm, embed_hbm, deg_out, emb_out,
                       didx, eidx, ones_v, gb0, gb1, bnc, deg_sh,
                       es0, es1, ws0, ws1):
    cid = lax.axis_index("c")
    sid = lax.axis_index("s")
    wid = sid * NC + cid

    def zb(i, carry):
        bnc[pl.ds(i * 16, 16)] = jnp.zeros((16,), jnp.float32)
        return carry
    lax.fori_loop(0, ROWS_PER_SUB // 16, zb, 0)
    pltpu.sync_copy(bnc, deg_sh.at[pl.ds(sid * ROWS_PER_SUB, ROWS_PER_SUB)])

    def ob(i, carry):
        ones_v[pl.ds(i * 16, 16)] = jnp.ones((16,), jnp.float32)
        return carry
    lax.fori_loop(0, CH // 16, ob, 0)
    plsc.subcore_barrier()

    pltpu.sync_copy(dst3_hbm.at[wid], didx)
    pltpu.sync_copy(xt3_hbm.at[wid], eidx)

    base = wid * (J_EMB * CHE)
    pltpu.async_copy(embed_hbm.at[eidx.at[0]], gb0, es0)
    pltpu.async_copy(embed_hbm.at[eidx.at[1]], gb1, es1)

    def ebody(j2, carry):
        for p, gb, es, ws in ((0, gb0, es0, ws0), (1, gb1, es1, ws1)):
            j = 2 * j2 + p
            pltpu.make_async_copy(embed_hbm.at[eidx.at[j]], gb, es).wait()
            pltpu.async_copy(gb, emb_out.at[pl.ds(base + j * CHE, CHE)], ws)
            pltpu.make_async_copy(
                gb, emb_out.at[pl.ds(base, CHE)], ws).wait()
            nj = jnp.minimum(j + 2, J_EMB - 1)
            pltpu.async_copy(embed_hbm.at[eidx.at[nj]], gb, es)
        return carry
    lax.fori_loop(0, J_EMB // 2, ebody, 0)
    pltpu.make_async_copy(embed_hbm.at[eidx.at[J_EMB - 1]], gb0, es0).wait()
    pltpu.make_async_copy(embed_hbm.at[eidx.at[J_EMB - 1]], gb1, es1).wait()

    def body(j, carry):
        pltpu.sync_copy(ones_v, deg_sh.at[didx.at[j]], add=True)
        return carry
    lax.fori_loop(0, J_CONV, body, 0)
    plsc.subcore_barrier()

    pltpu.sync_copy(deg_sh.at[pl.ds(sid * ROWS_PER_SUB, ROWS_PER_SUB)], bnc)
    pltpu.sync_copy(bnc, deg_out.at[cid, pl.ds(sid * ROWS_PER_SUB, ROWS_PER_SUB)])


@functools.cache
def _build_sc_conv():
    return functools.partial(
        pl.kernel,
        out_type=jax.ShapeDtypeStruct((NC, NPAD, H), jnp.float32),
        mesh=_mesh(),
        scratch_types=[
            pltpu.VMEM((J_CONV, CH), jnp.int32),
            pltpu.VMEM((J_CONV, CH), jnp.int32),
            pltpu.VMEM((CH, H), jnp.float32),
            pltpu.VMEM((CH, H), jnp.float32),
            pltpu.VMEM((ROWS_PER_SUB, H), jnp.float32),
            pltpu.VMEM_SHARED((NPAD, H), jnp.float32),
            pltpu.SemaphoreType.DMA,
            pltpu.SemaphoreType.DMA,
        ],
        compiler_params=pltpu.CompilerParams(use_tc_tiling_on_sc=False),
    )(_sc_conv_body)


def _sc_conv(src3, dst3, g):
    return _build_sc_conv()(src3, dst3, g)


def _sc_conv_body(src3_hbm, dst3_hbm, g_hbm, part_out,
                  sidx, didx, buf0, buf1, zbuf, agg_sh, sem0, sem1):
    cid = lax.axis_index("c")
    sid = lax.axis_index("s")
    wid = sid * NC + cid

    def zb(i, carry):
        r = i // 4
        k = i % 4
        zbuf[r, pl.ds(k * 16, 16)] = jnp.zeros((16,), jnp.float32)
        return carry
    lax.fori_loop(0, ROWS_PER_SUB * 4, zb, 0)
    pltpu.sync_copy(zbuf, agg_sh.at[pl.ds(sid * ROWS_PER_SUB, ROWS_PER_SUB)])
    plsc.subcore_barrier()

    pltpu.sync_copy(src3_hbm.at[wid], sidx)
    pltpu.sync_copy(dst3_hbm.at[wid], didx)

    pltpu.async_copy(g_hbm.at[sidx.at[0]], buf0, sem0)
    pltpu.async_copy(g_hbm.at[sidx.at[1]], buf1, sem1)

    def body(j2, carry):
        for p, buf, sem in ((0, buf0, sem0), (1, buf1, sem1)):
            j = 2 * j2 + p
            pltpu.make_async_copy(g_hbm.at[sidx.at[j]], buf, sem).wait()
            pltpu.sync_copy(buf, agg_sh.at[didx.at[j]], add=True)
            nj = jnp.minimum(j + 2, J_CONV - 1)
            pltpu.async_copy(g_hbm.at[sidx.at[nj]], buf, sem)
        return carry
    lax.fori_loop(0, J_CONV // 2, body, 0)
    pltpu.make_async_copy(g_hbm.at[sidx.at[J_CONV - 1]], buf0, sem0).wait()
    pltpu.make_async_copy(g_hbm.at[sidx.at[J_CONV - 1]], buf1, sem1).wait()
    plsc.subcore_barrier()

    pltpu.sync_copy(agg_sh.at[pl.ds(sid * ROWS_PER_SUB, ROWS_PER_SUB)], zbuf)
    pltpu.sync_copy(zbuf, part_out.at[cid, pl.ds(sid * ROWS_PER_SUB, ROWS_PER_SUB)])


@functools.cache
def _build_sc_readout():
    return functools.partial(
        pl.kernel,
        out_type=jax.ShapeDtypeStruct((NCHUNK_RO, CHR), jnp.float32),
        mesh=_mesh(),
        scratch_types=[
            pltpu.VMEM((J_RO, CH), jnp.int32),
            pltpu.VMEM((J_RO, CH), jnp.int32),
            pltpu.VMEM((CH, 2 * H), jnp.float32),
            pltpu.VMEM((CH, 2 * H), jnp.float32),
            pltpu.VMEM((CH, 2 * H), jnp.float32),
            pltpu.VMEM((CH, 2 * H), jnp.float32),
            pltpu.VMEM((CH,), jnp.float32),
            pltpu.VMEM((CH,), jnp.float32),
            pltpu.VMEM((2 * H,), jnp.float32),
            pltpu.SemaphoreType.DMA,
            pltpu.SemaphoreType.DMA,
            pltpu.SemaphoreType.DMA,
            pltpu.SemaphoreType.DMA,
        ],
        compiler_params=pltpu.CompilerParams(
            use_tc_tiling_on_sc=False, needs_layout_passes=False),
    )(_sc_readout_body)


def _sc_readout(src3, dst3, at1, bt2, wcat):
    return _build_sc_readout()(src3, dst3, at1, bt2, wcat)


def _sc_readout_body(src3_hbm, dst3_hbm, at1_hbm, bt2_hbm, wcat_hbm,
                     delta_out, sidx, didx, ba0, bb0, ba1, bb1, db0, db1,
                     wv, gs0, gs1, ws0, ws1):
    cid = lax.axis_index("c")
    sid = lax.axis_index("s")
    wid = sid * NC + cid
    base_row = wid * J_RO

    pltpu.sync_copy(src3_hbm.at[wid], sidx)
    pltpu.sync_copy(dst3_hbm.at[wid], didx)
    pltpu.sync_copy(wcat_hbm, wv)
    wregs = [wv[pl.ds(k * 16, 16)] for k in range(2 * H // 16)]
    lane = lax.iota(jnp.int32, 16)
    masks = [lane == e for e in range(16)]
    shuf = [lane ^ (1 << b) for b in range(4)]

    dnums = lax.GatherDimensionNumbers(
        offset_dims=(), collapsed_slice_dims=(0,), start_index_map=(0,))

    def _lanesum(v):
        for s in shuf:
            v = v + lax.gather(
                v, s[:, None], dimension_numbers=dnums, slice_sizes=(1,),
                mode=lax.GatherScatterMode.PROMISE_IN_BOUNDS)
        return v

    pltpu.async_copy(at1_hbm.at[sidx.at[0]], ba0, gs0)
    pltpu.async_copy(bt2_hbm.at[didx.at[0]], bb0, gs0)
    pltpu.async_copy(at1_hbm.at[sidx.at[1]], ba1, gs1)
    pltpu.async_copy(bt2_hbm.at[didx.at[1]], bb1, gs1)

    def body(j2, carry):
        for p, ba, bb, db, gs, ws in (
                (0, ba0, bb0, db0, gs0, ws0), (1, ba1, bb1, db1, gs1, ws1)):
            j = 2 * j2 + p
            pltpu.make_async_copy(at1_hbm.at[sidx.at[j]], ba, gs).wait()
            pltpu.make_async_copy(bt2_hbm.at[didx.at[j]], bb, gs).wait()

            @pl.when(j2 >= 1)
            def _():
                pltpu.make_async_copy(
                    db.at[pl.ds(0, CHR)], delta_out.at[base_row], ws).wait()

            def edge_grp(g, c2):
                acc16 = jnp.zeros((16,), jnp.float32)
                for e in range(16):
                    i = g * 16 + e
                    acc = jnp.maximum(
                        ba[i, pl.ds(0, 16)] + bb[i, pl.ds(0, 16)],
                        0.0) * wregs[0]
                    for k in range(1, 2 * H // 16):
                        acc = acc + jnp.maximum(
                            ba[i, pl.ds(k * 16, 16)]
                            + bb[i, pl.ds(k * 16, 16)], 0.0) * wregs[k]
                    acc16 = jnp.where(masks[e], _lanesum(acc), acc16)
                db[pl.ds(g * 16, 16)] = acc16
                return c2
            lax.fori_loop(0, CH // 16, edge_grp, 0)

            nj = jnp.minimum(j + 2, J_RO - 1)
            pltpu.async_copy(at1_hbm.at[sidx.at[nj]], ba, gs)
            pltpu.async_copy(bt2_hbm.at[didx.at[nj]], bb, gs)
            pltpu.async_copy(
                db.at[pl.ds(0, CHR)], delta_out.at[base_row + j], ws)
        return carry
    lax.fori_loop(0, J_RO // 2, body, 0)

    for ba, bb, db, gs, ws in (
            (ba0, bb0, db0, gs0, ws0), (ba1, bb1, db1, gs1, ws1)):
        pltpu.make_async_copy(at1_hbm.at[sidx.at[J_RO - 1]], ba, gs).wait()
        pltpu.make_async_copy(bt2_hbm.at[didx.at[J_RO - 1]], bb, gs).wait()
        pltpu.make_async_copy(
            db.at[pl.ds(0, CHR)], delta_out.at[base_row], ws).wait()


R = 1000


def _dis(deg_ref):
    deg = deg_ref[0, :, 0] + deg_ref[1, :, 0]
    return jnp.where(deg > 0, 1.0 / jnp.sqrt(deg), 0.0)


def _tc_g1_body(x_ref, w_ref, deg_ref, o_ref):
    dis = _dis(deg_ref)
    xw = jnp.dot(x_ref[...], w_ref[...], preferred_element_type=jnp.float32)
    o_ref[...] = xw * dis[:, None]


def _tc_g2_body(p_ref, deg_ref, b_ref, w_ref, o_ref):
    dis = _dis(deg_ref)
    agg = p_ref[0] + p_ref[1]
    h1 = jnp.maximum(agg * dis[:, None] + b_ref[...], 0.0)
    o_ref[...] = jnp.dot(
        h1, w_ref[...], preferred_element_type=jnp.float32) * dis[:, None]


def _tc_h2_body(p_ref, deg_ref, b_ref, o_ref):
    dis = _dis(deg_ref)
    agg = p_ref[0] + p_ref[1]
    o_ref[...] = jnp.maximum(agg * dis[:, None] + b_ref[...], 0.0)


def _tc_gru_body(e_ref, h_ref, wir, wiz, win, whr, whz, whn,
                 bir, biz, bin_, bhr, bhz, bhn, o_ref):
    def step(t, h):
        xt = e_ref[t]
        mm = lambda a, w: jnp.dot(a, w[...], preferred_element_type=jnp.float32)
        r = jax.nn.sigmoid(mm(xt, wir) + bir[...] + mm(h, whr) + bhr[...])
        z = jax.nn.sigmoid(mm(xt, wiz) + biz[...] + mm(h, whz) + bhz[...])
        n = jnp.tanh(mm(xt, win) + bin_[...] + r * (mm(h, whn) + bhn[...]))
        return (1.0 - z) * n + z * h

    o_ref[...] = lax.fori_loop(0, L // 2, step, h_ref[...])


def _tc_tables_body(h_ref, t_ref, w1a, w1b, wta, wtb, b1, bt,
                    at1_ref, bt2_ref):
    mm = lambda a, w: jnp.dot(a, w[...], preferred_element_type=jnp.float32)
    a = mm(h_ref[...], w1a) + b1[...]
    b = mm(h_ref[...], w1b)
    t1 = mm(t_ref[...], wta) + bt[...]
    t2 = mm(t_ref[...], wtb)
    at1_ref[...] = jnp.concatenate([a, t1], axis=1)
    bt2_ref[...] = jnp.concatenate([b, t2], axis=1)


RFIN = 256


def _tc_final_body(d_ref, bd_ref, o0_ref, o1_ref):
    delta = d_ref[...] + bd_ref[0, 0]
    def nsp(x):
        return -(jnp.maximum(x, 0.0) + jnp.log1p(jnp.exp(-jnp.abs(x))))
    o0_ref[...] = nsp(-delta)
    o1_ref[...] = nsp(delta)


def _deg_spec():
    return pl.BlockSpec((NC, R, 1), lambda i: (0, i, 0))


def _full(shape):
    return pl.BlockSpec(shape, lambda i: tuple(0 for _ in shape))


_g1_call = pl.pallas_call(
    _tc_g1_body,
    grid=(N // R,),
    in_specs=[pl.BlockSpec((R, D), lambda i: (i, 0)),
              _full((D, H)),
              _deg_spec()],
    out_specs=pl.BlockSpec((R, H), lambda i: (i, 0)),
    out_shape=jax.ShapeDtypeStruct((N, H), jnp.float32),
)

_g2_call = pl.pallas_call(
    _tc_g2_body,
    grid=(N // R,),
    in_specs=[pl.BlockSpec((NC, R, H), lambda i: (0, i, 0)),
              _deg_spec(),
              _full((1, H)),
              _full((H, H))],
    out_specs=pl.BlockSpec((R, H), lambda i: (i, 0)),
    out_shape=jax.ShapeDtypeStruct((N, H), jnp.float32),
)

_h2_call = pl.pallas_call(
    _tc_h2_body,
    grid=(N // R,),
    in_specs=[pl.BlockSpec((NC, R, H), lambda i: (0, i, 0)),
              _deg_spec(),
              _full((1, H))],
    out_specs=pl.BlockSpec((R, H), lambda i: (i, 0)),
    out_shape=jax.ShapeDtypeStruct((N, H), jnp.float32),
)

_gru_call = pl.pallas_call(
    _tc_gru_body,
    grid=(N // R,),
    in_specs=[pl.BlockSpec((L // 2, R, TF), lambda i: (0, i, 0)),
              pl.BlockSpec((R, H), lambda i: (i, 0))]
    + [_full((TF, H))] * 3 + [_full((H, H))] * 3 + [_full((1, H))] * 6,
    out_specs=pl.BlockSpec((R, H), lambda i: (i, 0)),
    out_shape=jax.ShapeDtypeStruct((N, H), jnp.float32),
)

_tables_call = pl.pallas_call(
    _tc_tables_body,
    grid=(N // R,),
    in_specs=[pl.BlockSpec((R, H), lambda i: (i, 0)),
              pl.BlockSpec((R, H), lambda i: (i, 0))]
    + [_full((H, H))] * 4 + [_full((1, H))] * 2,
    out_specs=[pl.BlockSpec((R, 2 * H), lambda i: (i, 0)),
               pl.BlockSpec((R, 2 * H), lambda i: (i, 0))],
    out_shape=[jax.ShapeDtypeStruct((N, 2 * H), jnp.float32),
               jax.ShapeDtypeStruct((N, 2 * H), jnp.float32)],
)


def _make_final_call(E):
    assert E == NCHUNK_RO * CHR and NCHUNK_RO % RFIN == 0
    return pl.pallas_call(
        _tc_final_body,
        grid=(NCHUNK_RO // RFIN,),
        in_specs=[pl.BlockSpec((RFIN, CHR), lambda i: (i, 0)),
                  _full((1, 1))],
        out_specs=[pl.BlockSpec((RFIN, CHR), lambda i: (i, 0)),
                   pl.BlockSpec((RFIN, CHR), lambda i: (i, 0))],
        out_shape=[jax.ShapeDtypeStruct((NCHUNK_RO, CHR), jnp.float32),
                   jax.ShapeDtypeStruct((NCHUNK_RO, CHR), jnp.float32)],
    )


def kernel(x, edge_index, xtext, conv1_W, conv1_b, conv2_W, conv2_b, embed,
           W_ih, W_hh, b_ih, b_hh, lin1_W, lin1_b, lint_W, lint_b,
           linf_W, linf_b):
    E = edge_index.shape[1]
    src0 = edge_index[0]
    dst0 = edge_index[1]
    loop = jnp.arange(N, dtype=jnp.int32)

    npad_c = EP_CONV - (E + N)
    pad_i = jnp.arange(npad_c, dtype=jnp.int32)
    src3c = jnp.concatenate([src0, loop, pad_i % N]).reshape(NW, J_CONV, CH)
    dst3c = jnp.concatenate(
        [dst0, loop, N + pad_i % (NPAD - N)]).reshape(NW, J_CONV, CH)

    fill = (jnp.arange(NCHUNK_RO * (CH - CHR), dtype=jnp.int32)
            % N).reshape(NCHUNK_RO, CH - CHR)
    src3r = jnp.concatenate(
        [src0.reshape(NCHUNK_RO, CHR), fill], axis=1).reshape(NW, J_RO, CH)
    dst3r = jnp.concatenate(
        [dst0.reshape(NCHUNK_RO, CHR), fill], axis=1).reshape(NW, J_RO, CH)

    xt3 = jnp.transpose(xtext).reshape(NW, J_EMB, CHE)

    deg2, emb_tm = _sc_deg_embed(dst3c, xt3, embed)
    deg3 = deg2.reshape(NC, NPAD, 1)

    emb3 = emb_tm.reshape(L, N, TF)
    wir, wiz, win = (W_ih[:H].T, W_ih[H:2 * H].T, W_ih[2 * H:].T)
    whr, whz, whn = (W_hh[:H].T, W_hh[H:2 * H].T, W_hh[2 * H:].T)
    bir, biz, bin_ = (b_ih[:H].reshape(1, H), b_ih[H:2 * H].reshape(1, H),
                      b_ih[2 * H:].reshape(1, H))
    bhr, bhz, bhn = (b_hh[:H].reshape(1, H), b_hh[H:2 * H].reshape(1, H),
                     b_hh[2 * H:].reshape(1, H))
    gru_w = (wir, wiz, win, whr, whz, whn, bir, biz, bin_, bhr, bhz, bhn)
    h0 = jnp.zeros((N, H), jnp.float32)

    g1 = _g1_call(x, conv1_W, deg3)
    p1 = _sc_conv(src3c, dst3c, g1)
    t10 = _gru_call(emb3[:L // 2], h0, *gru_w)
    t = _gru_call(emb3[L // 2:], t10, *gru_w)
    g2 = _g2_call(p1, deg3, conv1_b.reshape(1, H), conv2_W)
    p2 = _sc_conv(src3c, dst3c, g2)
    h2 = _h2_call(p2, deg3, conv2_b.reshape(1, H))

    at1, bt2 = _tables_call(
        h2, t, lin1_W[:H], lin1_W[H:], lint_W[:H], lint_W[H:],
        lin1_b.reshape(1, H), lint_b.reshape(1, H))

    wcat = linf_W[:, 0] - linf_W[:, 1]
    delta = _sc_readout(src3r, dst3r, at1, bt2, wcat)

    bd = (linf_b[0] - linf_b[1]).reshape(1, 1)
    o0, o1 = _make_final_call(E)(delta, bd)
    return jnp.stack([o0.reshape(E), o1.reshape(E)], axis=1)


# speedup vs baseline: 1.0815x; 1.0815x over previous
import functools

import jax
import jax.numpy as jnp
import numpy as np
from jax import lax
from jax.experimental import pallas as pl
from jax.experimental.pallas import tpu as pltpu
from jax.experimental.pallas import tpu_sc as plsc

N = 10000
D = 128
H = 64
TF = 64
L = 20
V = 100000
C = 2

NC = 2
NS = 16
NW = NC * NS
CH = 128

NPAD = 10240
ROWS_PER_SUB = NPAD // NS

J_CONV = 82
EP_CONV = NW * J_CONV * CH
CHR = 125
J_RO = 80
NCHUNK_RO = NW * J_RO
CHE = 125
J_EMB = 50
EP_EMB = NW * J_EMB * CHE
assert EP_EMB == N * L


def _mesh():
    return plsc.VectorSubcoreMesh(
        core_axis_name="c", subcore_axis_name="s",
        num_cores=NC, num_subcores=NS)


@functools.cache
def _build_sc_deg_embed():
    return functools.partial(
        pl.kernel,
        out_type=[
            jax.ShapeDtypeStruct((NC, NPAD), jnp.float32),
            jax.ShapeDtypeStruct((EP_EMB, TF), jnp.float32),
        ],
        mesh=_mesh(),
        scratch_types=[
            pltpu.VMEM((J_CONV, CH), jnp.int32),
            pltpu.VMEM((J_EMB, CHE), jnp.int32),
            pltpu.VMEM((CH,), jnp.float32),
            pltpu.VMEM((CHE, TF), jnp.float32),
            pltpu.VMEM((CHE, TF), jnp.float32),
            pltpu.VMEM((ROWS_PER_SUB,), jnp.float32),
            pltpu.VMEM_SHARED((NPAD,), jnp.float32),
            pltpu.SemaphoreType.DMA,
            pltpu.SemaphoreType.DMA,
            pltpu.SemaphoreType.DMA,
            pltpu.SemaphoreType.DMA,
        ],
        compiler_params=pltpu.CompilerParams(use_tc_tiling_on_sc=False),
    )(_sc_deg_embed_body)


def _sc_deg_embed(dst3, xt3, embed):
    return _build_sc_deg_embed()(dst3, xt3, embed)


def _sc_deg_embed_body(dst3_hbm, xt3_hbm, embed_hbm, deg_out, emb_out,
                       didx, eidx, ones_v, gb0, gb1, bnc, deg_sh,
                       es0, es1, ws0, ws1):
    cid = lax.axis_index("c")
    sid = lax.axis_index("s")
    wid = sid * NC + cid

    def zb(i, carry):
        bnc[pl.ds(i * 16, 16)] = jnp.zeros((16,), jnp.float32)
        return carry
    lax.fori_loop(0, ROWS_PER_SUB // 16, zb, 0)
    pltpu.sync_copy(bnc, deg_sh.at[pl.ds(sid * ROWS_PER_SUB, ROWS_PER_SUB)])

    def ob(i, carry):
        ones_v[pl.ds(i * 16, 16)] = jnp.ones((16,), jnp.float32)
        return carry
    lax.fori_loop(0, CH // 16, ob, 0)
    plsc.subcore_barrier()

    pltpu.sync_copy(dst3_hbm.at[wid], didx)
    pltpu.sync_copy(xt3_hbm.at[wid], eidx)

    base = wid * (J_EMB * CHE)
    pltpu.async_copy(embed_hbm.at[eidx.at[0]], gb0, es0)
    pltpu.async_copy(embed_hbm.at[eidx.at[1]], gb1, es1)

    def ebody(j2, carry):
        for p, gb, es, ws in ((0, gb0, es0, ws0), (1, gb1, es1, ws1)):
            j = 2 * j2 + p
            pltpu.make_async_copy(embed_hbm.at[eidx.at[j]], gb, es).wait()
            pltpu.async_copy(gb, emb_out.at[pl.ds(base + j * CHE, CHE)], ws)
            pltpu.make_async_copy(
                gb, emb_out.at[pl.ds(base, CHE)], ws).wait()
            nj = jnp.minimum(j + 2, J_EMB - 1)
            pltpu.async_copy(embed_hbm.at[eidx.at[nj]], gb, es)
        return carry
    lax.fori_loop(0, J_EMB // 2, ebody, 0)
    pltpu.make_async_copy(embed_hbm.at[eidx.at[J_EMB - 1]], gb0, es0).wait()
    pltpu.make_async_copy(embed_hbm.at[eidx.at[J_EMB - 1]], gb1, es1).wait()

    def body(j, carry):
        pltpu.sync_copy(ones_v, deg_sh.at[didx.at[j]], add=True)
        return carry
    lax.fori_loop(0, J_CONV, body, 0)
    plsc.subcore_barrier()

    pltpu.sync_copy(deg_sh.at[pl.ds(sid * ROWS_PER_SUB, ROWS_PER_SUB)], bnc)
    pltpu.sync_copy(bnc, deg_out.at[cid, pl.ds(sid * ROWS_PER_SUB, ROWS_PER_SUB)])


@functools.cache
def _build_sc_conv():
    return functools.partial(
        pl.kernel,
        out_type=jax.ShapeDtypeStruct((NC, NPAD, H), jnp.float32),
        mesh=_mesh(),
        scratch_types=[
            pltpu.VMEM((J_CONV, CH), jnp.int32),
            pltpu.VMEM((J_CONV, CH), jnp.int32),
            pltpu.VMEM((CH, H), jnp.float32),
            pltpu.VMEM((CH, H), jnp.float32),
            pltpu.VMEM((ROWS_PER_SUB, H), jnp.float32),
            pltpu.VMEM_SHARED((NPAD, H), jnp.float32),
            pltpu.SemaphoreType.DMA,
            pltpu.SemaphoreType.DMA,
        ],
        compiler_params=pltpu.CompilerParams(use_tc_tiling_on_sc=False),
    )(_sc_conv_body)


def _sc_conv(src3, dst3, g):
    return _build_sc_conv()(src3, dst3, g)


def _sc_conv_body(src3_hbm, dst3_hbm, g_hbm, part_out,
                  sidx, didx, buf0, buf1, zbuf, agg_sh, sem0, sem1):
    cid = lax.axis_index("c")
    sid = lax.axis_index("s")
    wid = sid * NC + cid

    def zb(i, carry):
        r = i // 4
        k = i % 4
        zbuf[r, pl.ds(k * 16, 16)] = jnp.zeros((16,), jnp.float32)
        return carry
    lax.fori_loop(0, ROWS_PER_SUB * 4, zb, 0)
    pltpu.sync_copy(zbuf, agg_sh.at[pl.ds(sid * ROWS_PER_SUB, ROWS_PER_SUB)])
    plsc.subcore_barrier()

    pltpu.sync_copy(src3_hbm.at[wid], sidx)
    pltpu.sync_copy(dst3_hbm.at[wid], didx)

    pltpu.async_copy(g_hbm.at[sidx.at[0]], buf0, sem0)
    pltpu.async_copy(g_hbm.at[sidx.at[1]], buf1, sem1)

    def body(j2, carry):
        for p, buf, sem in ((0, buf0, sem0), (1, buf1, sem1)):
            j = 2 * j2 + p
            pltpu.make_async_copy(g_hbm.at[sidx.at[j]], buf, sem).wait()
            pltpu.sync_copy(buf, agg_sh.at[didx.at[j]], add=True)
            nj = jnp.minimum(j + 2, J_CONV - 1)
            pltpu.async_copy(g_hbm.at[sidx.at[nj]], buf, sem)
        return carry
    lax.fori_loop(0, J_CONV // 2, body, 0)
    pltpu.make_async_copy(g_hbm.at[sidx.at[J_CONV - 1]], buf0, sem0).wait()
    pltpu.make_async_copy(g_hbm.at[sidx.at[J_CONV - 1]], buf1, sem1).wait()
    plsc.subcore_barrier()

    pltpu.sync_copy(agg_sh.at[pl.ds(sid * ROWS_PER_SUB, ROWS_PER_SUB)], zbuf)
    pltpu.sync_copy(zbuf, part_out.at[cid, pl.ds(sid * ROWS_PER_SUB, ROWS_PER_SUB)])


@functools.cache
def _build_sc_readout():
    return functools.partial(
        pl.kernel,
        out_type=jax.ShapeDtypeStruct((NCHUNK_RO, CHR), jnp.float32),
        mesh=_mesh(),
        scratch_types=[
            pltpu.VMEM((J_RO, CH), jnp.int32),
            pltpu.VMEM((J_RO, CH), jnp.int32),
            pltpu.VMEM((CH, 2 * H), jnp.float32),
            pltpu.VMEM((CH, 2 * H), jnp.float32),
            pltpu.VMEM((CH, 2 * H), jnp.float32),
            pltpu.VMEM((CH, 2 * H), jnp.float32),
            pltpu.VMEM((CH,), jnp.float32),
            pltpu.VMEM((CH,), jnp.float32),
            pltpu.VMEM((2 * H,), jnp.float32),
            pltpu.SemaphoreType.DMA,
            pltpu.SemaphoreType.DMA,
            pltpu.SemaphoreType.DMA,
            pltpu.SemaphoreType.DMA,
        ],
        compiler_params=pltpu.CompilerParams(
            use_tc_tiling_on_sc=False, needs_layout_passes=False),
    )(_sc_readout_body)


def _sc_readout(src3, dst3, at1, bt2, wcat):
    return _build_sc_readout()(src3, dst3, at1, bt2, wcat)


def _sc_readout_body(src3_hbm, dst3_hbm, at1_hbm, bt2_hbm, wcat_hbm,
                     delta_out, sidx, didx, ba0, bb0, ba1, bb1, db0, db1,
                     wv, gs0, gs1, ws0, ws1):
    cid = lax.axis_index("c")
    sid = lax.axis_index("s")
    wid = sid * NC + cid
    base_row = wid * J_RO

    pltpu.sync_copy(src3_hbm.at[wid], sidx)
    pltpu.sync_copy(dst3_hbm.at[wid], didx)
    pltpu.sync_copy(wcat_hbm, wv)
    wregs = [wv[pl.ds(k * 16, 16)] for k in range(2 * H // 16)]
    lane = lax.iota(jnp.int32, 16)
    masks = [lane == e for e in range(16)]
    shuf = [lane ^ (1 << b) for b in range(4)]

    dnums = lax.GatherDimensionNumbers(
        offset_dims=(), collapsed_slice_dims=(0,), start_index_map=(0,))

    def _lanesum(v):
        for s in shuf:
            v = v + lax.gather(
                v, s[:, None], dimension_numbers=dnums, slice_sizes=(1,),
                mode=lax.GatherScatterMode.PROMISE_IN_BOUNDS)
        return v

    pltpu.async_copy(at1_hbm.at[sidx.at[0]], ba0, gs0)
    pltpu.async_copy(bt2_hbm.at[didx.at[0]], bb0, gs0)
    pltpu.async_copy(at1_hbm.at[sidx.at[1]], ba1, gs1)
    pltpu.async_copy(bt2_hbm.at[didx.at[1]], bb1, gs1)

    def body(j2, carry):
        for p, ba, bb, db, gs, ws in (
                (0, ba0, bb0, db0, gs0, ws0), (1, ba1, bb1, db1, gs1, ws1)):
            j = 2 * j2 + p
            pltpu.make_async_copy(at1_hbm.at[sidx.at[j]], ba, gs).wait()
            pltpu.make_async_copy(bt2_hbm.at[didx.at[j]], bb, gs).wait()

            @pl.when(j2 >= 1)
            def _():
                pltpu.make_async_copy(
                    db.at[pl.ds(0, CHR)], delta_out.at[base_row], ws).wait()

            def edge_grp(g, c2):
                acc16 = jnp.zeros((16,), jnp.float32)
                for e in range(16):
                    i = g * 16 + e
                    acc = jnp.maximum(
                        ba[i, pl.ds(0, 16)] + bb[i, pl.ds(0, 16)],
                        0.0) * wregs[0]
                    for k in range(1, 2 * H // 16):
                        acc = acc + jnp.maximum(
                            ba[i, pl.ds(k * 16, 16)]
                            + bb[i, pl.ds(k * 16, 16)], 0.0) * wregs[k]
                    acc16 = jnp.where(masks[e], _lanesum(acc), acc16)
                db[pl.ds(g * 16, 16)] = acc16
                return c2
            lax.fori_loop(0, CH // 16, edge_grp, 0)

            nj = jnp.minimum(j + 2, J_RO - 1)
            pltpu.async_copy(at1_hbm.at[sidx.at[nj]], ba, gs)
            pltpu.async_copy(bt2_hbm.at[didx.at[nj]], bb, gs)
            pltpu.async_copy(
                db.at[pl.ds(0, CHR)], delta_out.at[base_row + j], ws)
        return carry
    lax.fori_loop(0, J_RO // 2, body, 0)

    for ba, bb, db, gs, ws in (
            (ba0, bb0, db0, gs0, ws0), (ba1, bb1, db1, gs1, ws1)):
        pltpu.make_async_copy(at1_hbm.at[sidx.at[J_RO - 1]], ba, gs).wait()
        pltpu.make_async_copy(bt2_hbm.at[didx.at[J_RO - 1]], bb, gs).wait()
        pltpu.make_async_copy(
            db.at[pl.ds(0, CHR)], delta_out.at[base_row], ws).wait()


R = 1000


def _dis(deg_ref):
    deg = deg_ref[0, :, 0] + deg_ref[1, :, 0]
    return jnp.where(deg > 0, 1.0 / jnp.sqrt(deg), 0.0)


def _tc_g1_body(x_ref, w_ref, deg_ref, o_ref):
    dis = _dis(deg_ref)
    xw = jnp.dot(x_ref[...], w_ref[...], preferred_element_type=jnp.float32)
    o_ref[...] = xw * dis[:, None]


def _tc_g2_body(p_ref, deg_ref, b_ref, w_ref, o_ref):
    dis = _dis(deg_ref)
    agg = p_ref[0] + p_ref[1]
    h1 = jnp.maximum(agg * dis[:, None] + b_ref[...], 0.0)
    o_ref[...] = jnp.dot(
        h1, w_ref[...], preferred_element_type=jnp.float32) * dis[:, None]


def _tc_h2_body(p_ref, deg_ref, b_ref, o_ref):
    dis = _dis(deg_ref)
    agg = p_ref[0] + p_ref[1]
    o_ref[...] = jnp.maximum(agg * dis[:, None] + b_ref[...], 0.0)


def _tc_gru_body(e_ref, h_ref, wir, wiz, win, whr, whz, whn,
                 bir, biz, bin_, bhr, bhz, bhn, o_ref):
    def step(t, h):
        xt = e_ref[t]
        mm = lambda a, w: jnp.dot(a, w[...], preferred_element_type=jnp.float32)
        r = jax.nn.sigmoid(mm(xt, wir) + bir[...] + mm(h, whr) + bhr[...])
        z = jax.nn.sigmoid(mm(xt, wiz) + biz[...] + mm(h, whz) + bhz[...])
        n = jnp.tanh(mm(xt, win) + bin_[...] + r * (mm(h, whn) + bhn[...]))
        return (1.0 - z) * n + z * h

    o_ref[...] = lax.fori_loop(0, L // 2, step, h_ref[...])


def _tc_tables_body(h_ref, t_ref, w1a, w1b, wta, wtb, b1, bt,
                    at1_ref, bt2_ref):
    mm = lambda a, w: jnp.dot(a, w[...], preferred_element_type=jnp.float32)
    a = mm(h_ref[...], w1a) + b1[...]
    b = mm(h_ref[...], w1b)
    t1 = mm(t_ref[...], wta) + bt[...]
    t2 = mm(t_ref[...], wtb)
    at1_ref[...] = jnp.concatenate([a, t1], axis=1)
    bt2_ref[...] = jnp.concatenate([b, t2], axis=1)


RFIN = 256


def _tc_final_body(d_ref, bd_ref, o0_ref, o1_ref):
    delta = d_ref[...] + bd_ref[0, 0]
    def nsp(x):
        return -(jnp.maximum(x, 0.0) + jnp.log1p(jnp.exp(-jnp.abs(x))))
    o0_ref[...] = nsp(-delta)
    o1_ref[...] = nsp(delta)


def _deg_spec():
    return pl.BlockSpec((NC, R, 1), lambda i: (0, i, 0))


def _full(shape):
    return pl.BlockSpec(shape, lambda i: tuple(0 for _ in shape))


_g1_call = pl.pallas_call(
    _tc_g1_body,
    grid=(N // R,),
    in_specs=[pl.BlockSpec((R, D), lambda i: (i, 0)),
              _full((D, H)),
              _deg_spec()],
    out_specs=pl.BlockSpec((R, H), lambda i: (i, 0)),
    out_shape=jax.ShapeDtypeStruct((N, H), jnp.float32),
)

_g2_call = pl.pallas_call(
    _tc_g2_body,
    grid=(N // R,),
    in_specs=[pl.BlockSpec((NC, R, H), lambda i: (0, i, 0)),
              _deg_spec(),
              _full((1, H)),
              _full((H, H))],
    out_specs=pl.BlockSpec((R, H), lambda i: (i, 0)),
    out_shape=jax.ShapeDtypeStruct((N, H), jnp.float32),
)

_h2_call = pl.pallas_call(
    _tc_h2_body,
    grid=(N // R,),
    in_specs=[pl.BlockSpec((NC, R, H), lambda i: (0, i, 0)),
              _deg_spec(),
              _full((1, H))],
    out_specs=pl.BlockSpec((R, H), lambda i: (i, 0)),
    out_shape=jax.ShapeDtypeStruct((N, H), jnp.float32),
)

_gru_call = pl.pallas_call(
    _tc_gru_body,
    grid=(N // R,),
    in_specs=[pl.BlockSpec((L // 2, R, TF), lambda i: (0, i, 0)),
              pl.BlockSpec((R, H), lambda i: (i, 0))]
    + [_full((TF, H))] * 3 + [_full((H, H))] * 3 + [_full((1, H))] * 6,
    out_specs=pl.BlockSpec((R, H), lambda i: (i, 0)),
    out_shape=jax.ShapeDtypeStruct((N, H), jnp.float32),
)

_tables_call = pl.pallas_call(
    _tc_tables_body,
    grid=(N // R,),
    in_specs=[pl.BlockSpec((R, H), lambda i: (i, 0)),
              pl.BlockSpec((R, H), lambda i: (i, 0))]
    + [_full((H, H))] * 4 + [_full((1, H))] * 2,
    out_specs=[pl.BlockSpec((R, 2 * H), lambda i: (i, 0)),
               pl.BlockSpec((R, 2 * H), lambda i: (i, 0))],
    out_shape=[jax.ShapeDtypeStruct((N, 2 * H), jnp.float32),
               jax.ShapeDtypeStruct((N, 2 * H), jnp.float32)],
)


def _make_final_call(E):
    assert E == NCHUNK_RO * CHR and NCHUNK_RO % RFIN == 0
    return pl.pallas_call(
        _tc_final_body,
        grid=(NCHUNK_RO // RFIN,),
        in_specs=[pl.BlockSpec((RFIN, CHR), lambda i: (i, 0)),
                  _full((1, 1))],
        out_specs=[pl.BlockSpec((RFIN, CHR), lambda i: (i, 0)),
                   pl.BlockSpec((RFIN, CHR), lambda i: (i, 0))],
        out_shape=[jax.ShapeDtypeStruct((NCHUNK_RO, CHR), jnp.float32),
                   jax.ShapeDtypeStruct((NCHUNK_RO, CHR), jnp.float32)],
    )


def kernel(x, edge_index, xtext, conv1_W, conv1_b, conv2_W, conv2_b, embed,
           W_ih, W_hh, b_ih, b_hh, lin1_W, lin1_b, lint_W, lint_b,
           linf_W, linf_b):
    E = edge_index.shape[1]
    src0 = edge_index[0]
    dst0 = edge_index[1]
    loop = jnp.asarray(np.arange(N, dtype=np.int32))

    npad_c = EP_CONV - (E + N)
    pad_src = jnp.asarray(np.arange(npad_c, dtype=np.int32) % N)
    pad_dst = jnp.asarray(
        N + np.arange(npad_c, dtype=np.int32) % (NPAD - N))
    src3c = jnp.concatenate([src0, loop, pad_src]).reshape(NW, J_CONV, CH)
    dst3c = jnp.concatenate([dst0, loop, pad_dst]).reshape(NW, J_CONV, CH)

    fill = jnp.asarray(
        (np.arange(NCHUNK_RO * (CH - CHR), dtype=np.int32)
         % N).reshape(NCHUNK_RO, CH - CHR))
    src3r = jnp.concatenate(
        [src0.reshape(NCHUNK_RO, CHR), fill], axis=1).reshape(NW, J_RO, CH)
    dst3r = jnp.concatenate(
        [dst0.reshape(NCHUNK_RO, CHR), fill], axis=1).reshape(NW, J_RO, CH)

    xt3 = jnp.transpose(xtext).reshape(NW, J_EMB, CHE)

    deg2, emb_tm = _sc_deg_embed(dst3c, xt3, embed)
    deg3 = deg2.reshape(NC, NPAD, 1)

    emb_a = emb_tm[:N * L // 2].reshape(L // 2, N, TF)
    emb_b = emb_tm[N * L // 2:].reshape(L // 2, N, TF)
    wir, wiz, win = (W_ih[:H].T, W_ih[H:2 * H].T, W_ih[2 * H:].T)
    whr, whz, whn = (W_hh[:H].T, W_hh[H:2 * H].T, W_hh[2 * H:].T)
    bir, biz, bin_ = (b_ih[:H].reshape(1, H), b_ih[H:2 * H].reshape(1, H),
                      b_ih[2 * H:].reshape(1, H))
    bhr, bhz, bhn = (b_hh[:H].reshape(1, H), b_hh[H:2 * H].reshape(1, H),
                     b_hh[2 * H:].reshape(1, H))
    gru_w = (wir, wiz, win, whr, whz, whn, bir, biz, bin_, bhr, bhz, bhn)
    h0 = jnp.zeros((N, H), jnp.float32)

    g1 = _g1_call(x, conv1_W, deg3)
    p1 = _sc_conv(src3c, dst3c, g1)
    t10 = _gru_call(emb_a, h0, *gru_w)
    t = _gru_call(emb_b, t10, *gru_w)
    g2 = _g2_call(p1, deg3, conv1_b.reshape(1, H), conv2_W)
    p2 = _sc_conv(src3c, dst3c, g2)
    h2 = _h2_call(p2, deg3, conv2_b.reshape(1, H))

    at1, bt2 = _tables_call(
        h2, t, lin1_W[:H], lin1_W[H:], lint_W[:H], lint_W[H:],
        lin1_b.reshape(1, H), lint_b.reshape(1, H))

    wcat = linf_W[:, 0] - linf_W[:, 1]
    delta = _sc_readout(src3r, dst3r, at1, bt2, wcat)

    bd = (linf_b[0] - linf_b[1]).reshape(1, 1)
    o0, o1 = _make_final_call(E)(delta, bd)
    return jnp.stack([o0.reshape(E), o1.reshape(E)], axis=1)


# speedup vs baseline: 1.1629x; 1.0753x over previous
import functools

import jax
import jax.numpy as jnp
import numpy as np
from jax import lax
from jax.experimental import pallas as pl
from jax.experimental.pallas import tpu as pltpu
from jax.experimental.pallas import tpu_sc as plsc

N = 10000
D = 128
H = 64
TF = 64
L = 20
V = 100000
C = 2

NC = 2
NS = 16
NW = NC * NS
CH = 128

NPAD = 10240
ROWS_PER_SUB = NPAD // NS

J_CONV = 82
EP_CONV = NW * J_CONV * CH
CHR = 125
J_RO = 80
NCHUNK_RO = NW * J_RO
CHE = 125
J_EMB = 50
EP_EMB = NW * J_EMB * CHE
assert EP_EMB == N * L


def _mesh():
    return plsc.VectorSubcoreMesh(
        core_axis_name="c", subcore_axis_name="s",
        num_cores=NC, num_subcores=NS)


@functools.cache
def _build_sc_deg_embed():
    return functools.partial(
        pl.kernel,
        out_type=[
            jax.ShapeDtypeStruct((NC, NPAD), jnp.float32),
            jax.ShapeDtypeStruct((EP_EMB, TF), jnp.float32),
        ],
        mesh=_mesh(),
        scratch_types=[
            pltpu.VMEM((J_CONV, CH), jnp.int32),
            pltpu.VMEM((J_EMB, CHE), jnp.int32),
            pltpu.VMEM((CH,), jnp.float32),
            pltpu.VMEM((CHE, TF), jnp.float32),
            pltpu.VMEM((CHE, TF), jnp.float32),
            pltpu.VMEM((ROWS_PER_SUB,), jnp.float32),
            pltpu.VMEM_SHARED((NPAD,), jnp.float32),
            pltpu.SemaphoreType.DMA,
            pltpu.SemaphoreType.DMA,
            pltpu.SemaphoreType.DMA,
            pltpu.SemaphoreType.DMA,
        ],
        compiler_params=pltpu.CompilerParams(use_tc_tiling_on_sc=False),
    )(_sc_deg_embed_body)


def _sc_deg_embed(dst3, xt3, embed):
    return _build_sc_deg_embed()(dst3, xt3, embed)


def _sc_deg_embed_body(dst3_hbm, xt3_hbm, embed_hbm, deg_out, emb_out,
                       didx, eidx, ones_v, gb0, gb1, bnc, deg_sh,
                       es0, es1, ws0, ws1):
    cid = lax.axis_index("c")
    sid = lax.axis_index("s")
    wid = sid * NC + cid

    def zb(i, carry):
        bnc[pl.ds(i * 16, 16)] = jnp.zeros((16,), jnp.float32)
        return carry
    lax.fori_loop(0, ROWS_PER_SUB // 16, zb, 0)
    pltpu.sync_copy(bnc, deg_sh.at[pl.ds(sid * ROWS_PER_SUB, ROWS_PER_SUB)])

    def ob(i, carry):
        ones_v[pl.ds(i * 16, 16)] = jnp.ones((16,), jnp.float32)
        return carry
    lax.fori_loop(0, CH // 16, ob, 0)
    plsc.subcore_barrier()

    pltpu.sync_copy(dst3_hbm.at[wid], didx)
    pltpu.sync_copy(xt3_hbm.at[wid], eidx)

    base = wid * (J_EMB * CHE)
    pltpu.async_copy(embed_hbm.at[eidx.at[0]], gb0, es0)
    pltpu.async_copy(embed_hbm.at[eidx.at[1]], gb1, es1)

    def ebody(j2, carry):
        for p, gb, es, ws in ((0, gb0, es0, ws0), (1, gb1, es1, ws1)):
            j = 2 * j2 + p
            pltpu.make_async_copy(embed_hbm.at[eidx.at[j]], gb, es).wait()
            pltpu.async_copy(gb, emb_out.at[pl.ds(base + j * CHE, CHE)], ws)
            pltpu.make_async_copy(
                gb, emb_out.at[pl.ds(base, CHE)], ws).wait()
            nj = jnp.minimum(j + 2, J_EMB - 1)
            pltpu.async_copy(embed_hbm.at[eidx.at[nj]], gb, es)
        return carry
    lax.fori_loop(0, J_EMB // 2, ebody, 0)
    pltpu.make_async_copy(embed_hbm.at[eidx.at[J_EMB - 1]], gb0, es0).wait()
    pltpu.make_async_copy(embed_hbm.at[eidx.at[J_EMB - 1]], gb1, es1).wait()

    def body(j, carry):
        pltpu.sync_copy(ones_v, deg_sh.at[didx.at[j]], add=True)
        return carry
    lax.fori_loop(0, J_CONV, body, 0)
    plsc.subcore_barrier()

    pltpu.sync_copy(deg_sh.at[pl.ds(sid * ROWS_PER_SUB, ROWS_PER_SUB)], bnc)
    pltpu.sync_copy(bnc, deg_out.at[cid, pl.ds(sid * ROWS_PER_SUB, ROWS_PER_SUB)])


@functools.cache
def _build_sc_conv():
    return functools.partial(
        pl.kernel,
        out_type=jax.ShapeDtypeStruct((NC, NPAD, H), jnp.float32),
        mesh=_mesh(),
        scratch_types=[
            pltpu.VMEM((J_CONV, CH), jnp.int32),
            pltpu.VMEM((J_CONV, CH), jnp.int32),
            pltpu.VMEM((CH, H), jnp.float32),
            pltpu.VMEM((CH, H), jnp.float32),
            pltpu.VMEM((ROWS_PER_SUB, H), jnp.float32),
            pltpu.VMEM_SHARED((NPAD, H), jnp.float32),
            pltpu.SemaphoreType.DMA,
            pltpu.SemaphoreType.DMA,
        ],
        compiler_params=pltpu.CompilerParams(use_tc_tiling_on_sc=False),
    )(_sc_conv_body)


def _sc_conv(src3, dst3, g):
    return _build_sc_conv()(src3, dst3, g)


def _sc_conv_body(src3_hbm, dst3_hbm, g_hbm, part_out,
                  sidx, didx, buf0, buf1, zbuf, agg_sh, sem0, sem1):
    cid = lax.axis_index("c")
    sid = lax.axis_index("s")
    wid = sid * NC + cid

    def zb(i, carry):
        r = i // 4
        k = i % 4
        zbuf[r, pl.ds(k * 16, 16)] = jnp.zeros((16,), jnp.float32)
        return carry
    lax.fori_loop(0, ROWS_PER_SUB * 4, zb, 0)
    pltpu.sync_copy(zbuf, agg_sh.at[pl.ds(sid * ROWS_PER_SUB, ROWS_PER_SUB)])
    plsc.subcore_barrier()

    pltpu.sync_copy(src3_hbm.at[wid], sidx)
    pltpu.sync_copy(dst3_hbm.at[wid], didx)

    pltpu.async_copy(g_hbm.at[sidx.at[0]], buf0, sem0)
    pltpu.async_copy(g_hbm.at[sidx.at[1]], buf1, sem1)

    def body(j2, carry):
        for p, buf, sem in ((0, buf0, sem0), (1, buf1, sem1)):
            j = 2 * j2 + p
            pltpu.make_async_copy(g_hbm.at[sidx.at[j]], buf, sem).wait()
            pltpu.sync_copy(buf, agg_sh.at[didx.at[j]], add=True)
            nj = jnp.minimum(j + 2, J_CONV - 1)
            pltpu.async_copy(g_hbm.at[sidx.at[nj]], buf, sem)
        return carry
    lax.fori_loop(0, J_CONV // 2, body, 0)
    pltpu.make_async_copy(g_hbm.at[sidx.at[J_CONV - 1]], buf0, sem0).wait()
    pltpu.make_async_copy(g_hbm.at[sidx.at[J_CONV - 1]], buf1, sem1).wait()
    plsc.subcore_barrier()

    pltpu.sync_copy(agg_sh.at[pl.ds(sid * ROWS_PER_SUB, ROWS_PER_SUB)], zbuf)
    pltpu.sync_copy(zbuf, part_out.at[cid, pl.ds(sid * ROWS_PER_SUB, ROWS_PER_SUB)])


@functools.cache
def _build_sc_readout():
    return functools.partial(
        pl.kernel,
        out_type=jax.ShapeDtypeStruct((NCHUNK_RO, CHR), jnp.float32),
        mesh=_mesh(),
        scratch_types=[
            pltpu.VMEM((J_RO, CH), jnp.int32),
            pltpu.VMEM((J_RO, CH), jnp.int32),
            pltpu.VMEM((CH, 2 * H), jnp.float32),
            pltpu.VMEM((CH, 2 * H), jnp.float32),
            pltpu.VMEM((CH, 2 * H), jnp.float32),
            pltpu.VMEM((CH, 2 * H), jnp.float32),
            pltpu.VMEM((CH,), jnp.float32),
            pltpu.VMEM((CH,), jnp.float32),
            pltpu.VMEM((2 * H,), jnp.float32),
            pltpu.SemaphoreType.DMA,
            pltpu.SemaphoreType.DMA,
            pltpu.SemaphoreType.DMA,
            pltpu.SemaphoreType.DMA,
        ],
        compiler_params=pltpu.CompilerParams(
            use_tc_tiling_on_sc=False, needs_layout_passes=False),
    )(_sc_readout_body)


def _sc_readout(src3, dst3, at1, bt2, wcat):
    return _build_sc_readout()(src3, dst3, at1, bt2, wcat)


def _sc_readout_body(src3_hbm, dst3_hbm, at1_hbm, bt2_hbm, wcat_hbm,
                     delta_out, sidx, didx, ba0, bb0, ba1, bb1, db0, db1,
                     wv, gs0, gs1, ws0, ws1):
    cid = lax.axis_index("c")
    sid = lax.axis_index("s")
    wid = sid * NC + cid
    base_row = wid * J_RO

    pltpu.sync_copy(src3_hbm.at[wid], sidx)
    pltpu.sync_copy(dst3_hbm.at[wid], didx)
    pltpu.sync_copy(wcat_hbm, wv)
    wregs = [wv[pl.ds(k * 16, 16)] for k in range(2 * H // 16)]
    lane = lax.iota(jnp.int32, 16)
    masks = [lane == e for e in range(16)]
    shuf = [lane ^ (1 << b) for b in range(4)]

    dnums = lax.GatherDimensionNumbers(
        offset_dims=(), collapsed_slice_dims=(0,), start_index_map=(0,))

    def _lanesum(v):
        for s in shuf:
            v = v + lax.gather(
                v, s[:, None], dimension_numbers=dnums, slice_sizes=(1,),
                mode=lax.GatherScatterMode.PROMISE_IN_BOUNDS)
        return v

    pltpu.async_copy(at1_hbm.at[sidx.at[0]], ba0, gs0)
    pltpu.async_copy(bt2_hbm.at[didx.at[0]], bb0, gs0)
    pltpu.async_copy(at1_hbm.at[sidx.at[1]], ba1, gs1)
    pltpu.async_copy(bt2_hbm.at[didx.at[1]], bb1, gs1)

    def body(j2, carry):
        for p, ba, bb, db, gs, ws in (
                (0, ba0, bb0, db0, gs0, ws0), (1, ba1, bb1, db1, gs1, ws1)):
            j = 2 * j2 + p
            pltpu.make_async_copy(at1_hbm.at[sidx.at[j]], ba, gs).wait()
            pltpu.make_async_copy(bt2_hbm.at[didx.at[j]], bb, gs).wait()

            @pl.when(j2 >= 1)
            def _():
                pltpu.make_async_copy(
                    db.at[pl.ds(0, CHR)], delta_out.at[base_row], ws).wait()

            def edge_grp(g, c2):
                acc16 = jnp.zeros((16,), jnp.float32)
                for e in range(16):
                    i = g * 16 + e
                    acc = jnp.maximum(
                        ba[i, pl.ds(0, 16)] + bb[i, pl.ds(0, 16)],
                        0.0) * wregs[0]
                    for k in range(1, 2 * H // 16):
                        acc = acc + jnp.maximum(
                            ba[i, pl.ds(k * 16, 16)]
                            + bb[i, pl.ds(k * 16, 16)], 0.0) * wregs[k]
                    acc16 = jnp.where(masks[e], _lanesum(acc), acc16)
                db[pl.ds(g * 16, 16)] = acc16
                return c2
            lax.fori_loop(0, CH // 16, edge_grp, 0)

            nj = jnp.minimum(j + 2, J_RO - 1)
            pltpu.async_copy(at1_hbm.at[sidx.at[nj]], ba, gs)
            pltpu.async_copy(bt2_hbm.at[didx.at[nj]], bb, gs)
            pltpu.async_copy(
                db.at[pl.ds(0, CHR)], delta_out.at[base_row + j], ws)
        return carry
    lax.fori_loop(0, J_RO // 2, body, 0)

    for ba, bb, db, gs, ws in (
            (ba0, bb0, db0, gs0, ws0), (ba1, bb1, db1, gs1, ws1)):
        pltpu.make_async_copy(at1_hbm.at[sidx.at[J_RO - 1]], ba, gs).wait()
        pltpu.make_async_copy(bt2_hbm.at[didx.at[J_RO - 1]], bb, gs).wait()
        pltpu.make_async_copy(
            db.at[pl.ds(0, CHR)], delta_out.at[base_row], ws).wait()


R = 1000


def _dis(deg_ref):
    deg = deg_ref[0, :, 0] + deg_ref[1, :, 0]
    return jnp.where(deg > 0, 1.0 / jnp.sqrt(deg), 0.0)


def _tc_g1_body(x_ref, w_ref, deg_ref, o_ref):
    dis = _dis(deg_ref)
    xw = jnp.dot(x_ref[...], w_ref[...], preferred_element_type=jnp.float32)
    o_ref[...] = xw * dis[:, None]


def _tc_g2_body(p_ref, deg_ref, b_ref, w_ref, o_ref):
    dis = _dis(deg_ref)
    agg = p_ref[0] + p_ref[1]
    h1 = jnp.maximum(agg * dis[:, None] + b_ref[...], 0.0)
    o_ref[...] = jnp.dot(
        h1, w_ref[...], preferred_element_type=jnp.float32) * dis[:, None]


def _tc_h2_body(p_ref, deg_ref, b_ref, o_ref):
    dis = _dis(deg_ref)
    agg = p_ref[0] + p_ref[1]
    o_ref[...] = jnp.maximum(agg * dis[:, None] + b_ref[...], 0.0)


def _tc_gru_body(e_ref, h_ref, wir, wiz, win, whr, whz, whn,
                 bir, biz, bin_, bhr, bhz, bhn, o_ref):
    def step(t, h):
        xt = e_ref[t]
        mm = lambda a, w: jnp.dot(a, w[...], preferred_element_type=jnp.float32)
        r = jax.nn.sigmoid(mm(xt, wir) + bir[...] + mm(h, whr) + bhr[...])
        z = jax.nn.sigmoid(mm(xt, wiz) + biz[...] + mm(h, whz) + bhz[...])
        n = jnp.tanh(mm(xt, win) + bin_[...] + r * (mm(h, whn) + bhn[...]))
        return (1.0 - z) * n + z * h

    o_ref[...] = lax.fori_loop(0, e_ref.shape[0], step, h_ref[...])


def _tc_tables_body(h_ref, t_ref, w1a, w1b, wta, wtb, b1, bt,
                    at1_ref, bt2_ref):
    mm = lambda a, w: jnp.dot(a, w[...], preferred_element_type=jnp.float32)
    a = mm(h_ref[...], w1a) + b1[...]
    b = mm(h_ref[...], w1b)
    t1 = mm(t_ref[...], wta) + bt[...]
    t2 = mm(t_ref[...], wtb)
    at1_ref[...] = jnp.concatenate([a, t1], axis=1)
    bt2_ref[...] = jnp.concatenate([b, t2], axis=1)


RFIN = 256


def _tc_final_body(d_ref, bd_ref, o0_ref, o1_ref):
    delta = d_ref[...] + bd_ref[0, 0]
    def nsp(x):
        return -(jnp.maximum(x, 0.0) + jnp.log1p(jnp.exp(-jnp.abs(x))))
    o0_ref[...] = nsp(-delta)
    o1_ref[...] = nsp(delta)


def _deg_spec():
    return pl.BlockSpec((NC, R, 1), lambda i: (0, i, 0))


def _full(shape):
    return pl.BlockSpec(shape, lambda i: tuple(0 for _ in shape))


_g1_call = pl.pallas_call(
    _tc_g1_body,
    grid=(N // R,),
    in_specs=[pl.BlockSpec((R, D), lambda i: (i, 0)),
              _full((D, H)),
              _deg_spec()],
    out_specs=pl.BlockSpec((R, H), lambda i: (i, 0)),
    out_shape=jax.ShapeDtypeStruct((N, H), jnp.float32),
)

_g2_call = pl.pallas_call(
    _tc_g2_body,
    grid=(N // R,),
    in_specs=[pl.BlockSpec((NC, R, H), lambda i: (0, i, 0)),
              _deg_spec(),
              _full((1, H)),
              _full((H, H))],
    out_specs=pl.BlockSpec((R, H), lambda i: (i, 0)),
    out_shape=jax.ShapeDtypeStruct((N, H), jnp.float32),
)

_h2_call = pl.pallas_call(
    _tc_h2_body,
    grid=(N // R,),
    in_specs=[pl.BlockSpec((NC, R, H), lambda i: (0, i, 0)),
              _deg_spec(),
              _full((1, H))],
    out_specs=pl.BlockSpec((R, H), lambda i: (i, 0)),
    out_shape=jax.ShapeDtypeStruct((N, H), jnp.float32),
)

def _make_gru_call(nsteps):
    return pl.pallas_call(
        _tc_gru_body,
        grid=(N // R,),
        in_specs=[pl.BlockSpec((nsteps, R, TF), lambda i: (0, i, 0)),
                  pl.BlockSpec((R, H), lambda i: (i, 0))]
        + [_full((TF, H))] * 3 + [_full((H, H))] * 3 + [_full((1, H))] * 6,
        out_specs=pl.BlockSpec((R, H), lambda i: (i, 0)),
        out_shape=jax.ShapeDtypeStruct((N, H), jnp.float32),
    )


_gru_call = _make_gru_call(L)

_tables_call = pl.pallas_call(
    _tc_tables_body,
    grid=(N // R,),
    in_specs=[pl.BlockSpec((R, H), lambda i: (i, 0)),
              pl.BlockSpec((R, H), lambda i: (i, 0))]
    + [_full((H, H))] * 4 + [_full((1, H))] * 2,
    out_specs=[pl.BlockSpec((R, 2 * H), lambda i: (i, 0)),
               pl.BlockSpec((R, 2 * H), lambda i: (i, 0))],
    out_shape=[jax.ShapeDtypeStruct((N, 2 * H), jnp.float32),
               jax.ShapeDtypeStruct((N, 2 * H), jnp.float32)],
)


def _make_final_call(E):
    assert E == NCHUNK_RO * CHR and NCHUNK_RO % RFIN == 0
    return pl.pallas_call(
        _tc_final_body,
        grid=(NCHUNK_RO // RFIN,),
        in_specs=[pl.BlockSpec((RFIN, CHR), lambda i: (i, 0)),
                  _full((1, 1))],
        out_specs=[pl.BlockSpec((RFIN, CHR), lambda i: (i, 0)),
                   pl.BlockSpec((RFIN, CHR), lambda i: (i, 0))],
        out_shape=[jax.ShapeDtypeStruct((NCHUNK_RO, CHR), jnp.float32),
                   jax.ShapeDtypeStruct((NCHUNK_RO, CHR), jnp.float32)],
    )


def kernel(x, edge_index, xtext, conv1_W, conv1_b, conv2_W, conv2_b, embed,
           W_ih, W_hh, b_ih, b_hh, lin1_W, lin1_b, lint_W, lint_b,
           linf_W, linf_b):
    E = edge_index.shape[1]
    src0 = edge_index[0]
    dst0 = edge_index[1]
    loop = jnp.asarray(np.arange(N, dtype=np.int32))

    npad_c = EP_CONV - (E + N)
    pad_src = jnp.asarray(np.arange(npad_c, dtype=np.int32) % N)
    pad_dst = jnp.asarray(
        N + np.arange(npad_c, dtype=np.int32) % (NPAD - N))
    src3c = jnp.concatenate([src0, loop, pad_src]).reshape(NW, J_CONV, CH)
    dst3c = jnp.concatenate([dst0, loop, pad_dst]).reshape(NW, J_CONV, CH)

    fill = jnp.asarray(
        (np.arange(NCHUNK_RO * (CH - CHR), dtype=np.int32)
         % N).reshape(NCHUNK_RO, CH - CHR))
    src3r = jnp.concatenate(
        [src0.reshape(NCHUNK_RO, CHR), fill], axis=1).reshape(NW, J_RO, CH)
    dst3r = jnp.concatenate(
        [dst0.reshape(NCHUNK_RO, CHR), fill], axis=1).reshape(NW, J_RO, CH)

    xt3 = jnp.transpose(xtext).reshape(NW, J_EMB, CHE)

    deg2, emb_tm = _sc_deg_embed(dst3c, xt3, embed)
    deg3 = deg2.reshape(NC, NPAD, 1)

    emb3 = emb_tm.reshape(L, N, TF)
    wir, wiz, win = (W_ih[:H].T, W_ih[H:2 * H].T, W_ih[2 * H:].T)
    whr, whz, whn = (W_hh[:H].T, W_hh[H:2 * H].T, W_hh[2 * H:].T)
    bir, biz, bin_ = (b_ih[:H].reshape(1, H), b_ih[H:2 * H].reshape(1, H),
                      b_ih[2 * H:].reshape(1, H))
    bhr, bhz, bhn = (b_hh[:H].reshape(1, H), b_hh[H:2 * H].reshape(1, H),
                     b_hh[2 * H:].reshape(1, H))
    gru_w = (wir, wiz, win, whr, whz, whn, bir, biz, bin_, bhr, bhz, bhn)
    h0 = jnp.zeros((N, H), jnp.float32)

    g1 = _g1_call(x, conv1_W, deg3)
    p1 = _sc_conv(src3c, dst3c, g1)
    t = _gru_call(emb3, h0, *gru_w)
    g2 = _g2_call(p1, deg3, conv1_b.reshape(1, H), conv2_W)
    p2 = _sc_conv(src3c, dst3c, g2)
    h2 = _h2_call(p2, deg3, conv2_b.reshape(1, H))

    at1, bt2 = _tables_call(
        h2, t, lin1_W[:H], lin1_W[H:], lint_W[:H], lint_W[H:],
        lin1_b.reshape(1, H), lint_b.reshape(1, H))

    wcat = linf_W[:, 0] - linf_W[:, 1]
    delta = _sc_readout(src3r, dst3r, at1, bt2, wcat)

    bd = (linf_b[0] - linf_b[1]).reshape(1, 1)
    o0, o1 = _make_final_call(E)(delta, bd)
    return jnp.stack([o0.reshape(E), o1.reshape(E)], axis=1)


# speedup vs baseline: 1.1647x; 1.0015x over previous
import functools

import jax
import jax.numpy as jnp
import numpy as np
from jax import lax
from jax.experimental import pallas as pl
from jax.experimental.pallas import tpu as pltpu
from jax.experimental.pallas import tpu_sc as plsc

N = 10000
D = 128
H = 64
TF = 64
L = 20
V = 100000
C = 2

NC = 2
NS = 16
NW = NC * NS
CH = 128

NPAD = 10240
ROWS_PER_SUB = NPAD // NS

J_CONV = 82
EP_CONV = NW * J_CONV * CH
CHR = 125
J_RO = 80
NCHUNK_RO = NW * J_RO
CHE = 125
J_EMB = 50
EP_EMB = NW * J_EMB * CHE
assert EP_EMB == N * L


def _mesh():
    return plsc.VectorSubcoreMesh(
        core_axis_name="c", subcore_axis_name="s",
        num_cores=NC, num_subcores=NS)


@functools.cache
def _build_sc_deg_embed():
    return functools.partial(
        pl.kernel,
        out_type=[
            jax.ShapeDtypeStruct((NC, NPAD), jnp.float32),
            jax.ShapeDtypeStruct((EP_EMB, TF), jnp.float32),
        ],
        mesh=_mesh(),
        scratch_types=[
            pltpu.VMEM((J_CONV, CH), jnp.int32),
            pltpu.VMEM((J_EMB, CHE), jnp.int32),
            pltpu.VMEM((CH,), jnp.float32),
            pltpu.VMEM((CHE, TF), jnp.float32),
            pltpu.VMEM((CHE, TF), jnp.float32),
            pltpu.VMEM((ROWS_PER_SUB,), jnp.float32),
            pltpu.VMEM_SHARED((NPAD,), jnp.float32),
            pltpu.SemaphoreType.DMA,
            pltpu.SemaphoreType.DMA,
            pltpu.SemaphoreType.DMA,
            pltpu.SemaphoreType.DMA,
        ],
        compiler_params=pltpu.CompilerParams(use_tc_tiling_on_sc=False),
    )(_sc_deg_embed_body)


def _sc_deg_embed(dst3, xt3, embed):
    return _build_sc_deg_embed()(dst3, xt3, embed)


def _sc_deg_embed_body(dst3_hbm, xt3_hbm, embed_hbm, deg_out, emb_out,
                       didx, eidx, ones_v, gb0, gb1, bnc, deg_sh,
                       es0, es1, ws0, ws1):
    cid = lax.axis_index("c")
    sid = lax.axis_index("s")
    wid = sid * NC + cid

    def zb(i, carry):
        bnc[pl.ds(i * 16, 16)] = jnp.zeros((16,), jnp.float32)
        return carry
    lax.fori_loop(0, ROWS_PER_SUB // 16, zb, 0)
    pltpu.sync_copy(bnc, deg_sh.at[pl.ds(sid * ROWS_PER_SUB, ROWS_PER_SUB)])

    def ob(i, carry):
        ones_v[pl.ds(i * 16, 16)] = jnp.ones((16,), jnp.float32)
        return carry
    lax.fori_loop(0, CH // 16, ob, 0)
    plsc.subcore_barrier()

    pltpu.sync_copy(dst3_hbm.at[wid], didx)
    pltpu.sync_copy(xt3_hbm.at[wid], eidx)

    base = wid * (J_EMB * CHE)
    pltpu.async_copy(embed_hbm.at[eidx.at[0]], gb0, es0)
    pltpu.async_copy(embed_hbm.at[eidx.at[1]], gb1, es1)

    def ebody(j2, carry):
        for p, gb, es, ws in ((0, gb0, es0, ws0), (1, gb1, es1, ws1)):
            j = 2 * j2 + p
            pltpu.make_async_copy(embed_hbm.at[eidx.at[j]], gb, es).wait()
            pltpu.async_copy(gb, emb_out.at[pl.ds(base + j * CHE, CHE)], ws)
            pltpu.make_async_copy(
                gb, emb_out.at[pl.ds(base, CHE)], ws).wait()
            nj = jnp.minimum(j + 2, J_EMB - 1)
            pltpu.async_copy(embed_hbm.at[eidx.at[nj]], gb, es)
        return carry
    lax.fori_loop(0, J_EMB // 2, ebody, 0)
    pltpu.make_async_copy(embed_hbm.at[eidx.at[J_EMB - 1]], gb0, es0).wait()
    pltpu.make_async_copy(embed_hbm.at[eidx.at[J_EMB - 1]], gb1, es1).wait()

    def body(j, carry):
        pltpu.sync_copy(ones_v, deg_sh.at[didx.at[j]], add=True)
        return carry
    lax.fori_loop(0, J_CONV, body, 0)
    plsc.subcore_barrier()

    pltpu.sync_copy(deg_sh.at[pl.ds(sid * ROWS_PER_SUB, ROWS_PER_SUB)], bnc)
    pltpu.sync_copy(bnc, deg_out.at[cid, pl.ds(sid * ROWS_PER_SUB, ROWS_PER_SUB)])


@functools.cache
def _build_sc_conv():
    return functools.partial(
        pl.kernel,
        out_type=jax.ShapeDtypeStruct((NC, NPAD, H), jnp.float32),
        mesh=_mesh(),
        scratch_types=[
            pltpu.VMEM((J_CONV, CH), jnp.int32),
            pltpu.VMEM((J_CONV, CH), jnp.int32),
            pltpu.VMEM((CH, H), jnp.float32),
            pltpu.VMEM((CH, H), jnp.float32),
            pltpu.VMEM((ROWS_PER_SUB, H), jnp.float32),
            pltpu.VMEM_SHARED((NPAD, H), jnp.float32),
            pltpu.SemaphoreType.DMA,
            pltpu.SemaphoreType.DMA,
        ],
        compiler_params=pltpu.CompilerParams(use_tc_tiling_on_sc=False),
    )(_sc_conv_body)


def _sc_conv(src3, dst3, g):
    return _build_sc_conv()(src3, dst3, g)


def _sc_conv_body(src3_hbm, dst3_hbm, g_hbm, part_out,
                  sidx, didx, buf0, buf1, zbuf, agg_sh, sem0, sem1):
    cid = lax.axis_index("c")
    sid = lax.axis_index("s")
    wid = sid * NC + cid

    def zb(i, carry):
        r = i // 4
        k = i % 4
        zbuf[r, pl.ds(k * 16, 16)] = jnp.zeros((16,), jnp.float32)
        return carry
    lax.fori_loop(0, ROWS_PER_SUB * 4, zb, 0)
    pltpu.sync_copy(zbuf, agg_sh.at[pl.ds(sid * ROWS_PER_SUB, ROWS_PER_SUB)])
    plsc.subcore_barrier()

    pltpu.sync_copy(src3_hbm.at[wid], sidx)
    pltpu.sync_copy(dst3_hbm.at[wid], didx)

    pltpu.async_copy(g_hbm.at[sidx.at[0]], buf0, sem0)
    pltpu.async_copy(g_hbm.at[sidx.at[1]], buf1, sem1)

    def body(j2, carry):
        for p, buf, sem in ((0, buf0, sem0), (1, buf1, sem1)):
            j = 2 * j2 + p
            pltpu.make_async_copy(g_hbm.at[sidx.at[j]], buf, sem).wait()
            pltpu.sync_copy(buf, agg_sh.at[didx.at[j]], add=True)
            nj = jnp.minimum(j + 2, J_CONV - 1)
            pltpu.async_copy(g_hbm.at[sidx.at[nj]], buf, sem)
        return carry
    lax.fori_loop(0, J_CONV // 2, body, 0)
    pltpu.make_async_copy(g_hbm.at[sidx.at[J_CONV - 1]], buf0, sem0).wait()
    pltpu.make_async_copy(g_hbm.at[sidx.at[J_CONV - 1]], buf1, sem1).wait()
    plsc.subcore_barrier()

    pltpu.sync_copy(agg_sh.at[pl.ds(sid * ROWS_PER_SUB, ROWS_PER_SUB)], zbuf)
    pltpu.sync_copy(zbuf, part_out.at[cid, pl.ds(sid * ROWS_PER_SUB, ROWS_PER_SUB)])


@functools.cache
def _build_sc_readout():
    return functools.partial(
        pl.kernel,
        out_type=jax.ShapeDtypeStruct((NCHUNK_RO, CHR), jnp.float32),
        mesh=_mesh(),
        scratch_types=[
            pltpu.VMEM((J_RO, CH), jnp.int32),
            pltpu.VMEM((J_RO, CH), jnp.int32),
            pltpu.VMEM((CH, 2 * H), jnp.float32),
            pltpu.VMEM((CH, 2 * H), jnp.float32),
            pltpu.VMEM((CH, 2 * H), jnp.float32),
            pltpu.VMEM((CH, 2 * H), jnp.float32),
            pltpu.VMEM((CH,), jnp.float32),
            pltpu.VMEM((CH,), jnp.float32),
            pltpu.VMEM((2 * H,), jnp.float32),
            pltpu.SemaphoreType.DMA,
            pltpu.SemaphoreType.DMA,
            pltpu.SemaphoreType.DMA,
            pltpu.SemaphoreType.DMA,
        ],
        compiler_params=pltpu.CompilerParams(
            use_tc_tiling_on_sc=False, needs_layout_passes=False),
    )(_sc_readout_body)


def _sc_readout(src3, dst3, at1, bt2, wcat):
    return _build_sc_readout()(src3, dst3, at1, bt2, wcat)


def _sc_readout_body(src3_hbm, dst3_hbm, at1_hbm, bt2_hbm, wcat_hbm,
                     delta_out, sidx, didx, ba0, bb0, ba1, bb1, db0, db1,
                     wv, gs0, gs1, ws0, ws1):
    cid = lax.axis_index("c")
    sid = lax.axis_index("s")
    wid = sid * NC + cid
    base_row = wid * J_RO

    pltpu.sync_copy(src3_hbm.at[wid], sidx)
    pltpu.sync_copy(dst3_hbm.at[wid], didx)
    pltpu.sync_copy(wcat_hbm, wv)
    wregs = [wv[pl.ds(k * 16, 16)] for k in range(2 * H // 16)]
    lane = lax.iota(jnp.int32, 16)
    masks = [lane == e for e in range(16)]
    shuf = [lane ^ (1 << b) for b in range(4)]

    dnums = lax.GatherDimensionNumbers(
        offset_dims=(), collapsed_slice_dims=(0,), start_index_map=(0,))

    def _lanesum(v):
        for s in shuf:
            v = v + lax.gather(
                v, s[:, None], dimension_numbers=dnums, slice_sizes=(1,),
                mode=lax.GatherScatterMode.PROMISE_IN_BOUNDS)
        return v

    pltpu.async_copy(at1_hbm.at[sidx.at[0]], ba0, gs0)
    pltpu.async_copy(bt2_hbm.at[didx.at[0]], bb0, gs0)
    pltpu.async_copy(at1_hbm.at[sidx.at[1]], ba1, gs1)
    pltpu.async_copy(bt2_hbm.at[didx.at[1]], bb1, gs1)

    def body(j2, carry):
        for p, ba, bb, db, gs, ws in (
                (0, ba0, bb0, db0, gs0, ws0), (1, ba1, bb1, db1, gs1, ws1)):
            j = 2 * j2 + p
            pltpu.make_async_copy(at1_hbm.at[sidx.at[j]], ba, gs).wait()
            pltpu.make_async_copy(bt2_hbm.at[didx.at[j]], bb, gs).wait()

            @pl.when(j2 >= 1)
            def _():
                pltpu.make_async_copy(
                    db.at[pl.ds(0, CHR)], delta_out.at[base_row], ws).wait()

            def edge_grp(g, c2):
                acc16 = jnp.zeros((16,), jnp.float32)
                for e in range(16):
                    i = g * 16 + e
                    acc = jnp.maximum(
                        ba[i, pl.ds(0, 16)] + bb[i, pl.ds(0, 16)],
                        0.0) * wregs[0]
                    for k in range(1, 2 * H // 16):
                        acc = acc + jnp.maximum(
                            ba[i, pl.ds(k * 16, 16)]
                            + bb[i, pl.ds(k * 16, 16)], 0.0) * wregs[k]
                    acc16 = jnp.where(masks[e], _lanesum(acc), acc16)
                db[pl.ds(g * 16, 16)] = acc16
                return c2
            lax.fori_loop(0, CH // 16, edge_grp, 0)

            nj = jnp.minimum(j + 2, J_RO - 1)
            pltpu.async_copy(at1_hbm.at[sidx.at[nj]], ba, gs)
            pltpu.async_copy(bt2_hbm.at[didx.at[nj]], bb, gs)
            pltpu.async_copy(
                db.at[pl.ds(0, CHR)], delta_out.at[base_row + j], ws)
        return carry
    lax.fori_loop(0, J_RO // 2, body, 0)

    for ba, bb, db, gs, ws in (
            (ba0, bb0, db0, gs0, ws0), (ba1, bb1, db1, gs1, ws1)):
        pltpu.make_async_copy(at1_hbm.at[sidx.at[J_RO - 1]], ba, gs).wait()
        pltpu.make_async_copy(bt2_hbm.at[didx.at[J_RO - 1]], bb, gs).wait()
        pltpu.make_async_copy(
            db.at[pl.ds(0, CHR)], delta_out.at[base_row], ws).wait()


R = 1000


def _dis(deg_ref):
    deg = deg_ref[0, :, 0] + deg_ref[1, :, 0]
    return jnp.where(deg > 0, 1.0 / jnp.sqrt(deg), 0.0)


def _tc_g1_body(x_ref, w_ref, deg_ref, o_ref):
    dis = _dis(deg_ref)
    xw = jnp.dot(x_ref[...], w_ref[...], preferred_element_type=jnp.float32)
    o_ref[...] = xw * dis[:, None]


def _tc_g2_body(p_ref, deg_ref, b_ref, w_ref, o_ref):
    dis = _dis(deg_ref)
    agg = p_ref[0] + p_ref[1]
    h1 = jnp.maximum(agg * dis[:, None] + b_ref[...], 0.0)
    o_ref[...] = jnp.dot(
        h1, w_ref[...], preferred_element_type=jnp.float32) * dis[:, None]


def _tc_h2_body(p_ref, deg_ref, b_ref, o_ref):
    dis = _dis(deg_ref)
    agg = p_ref[0] + p_ref[1]
    o_ref[...] = jnp.maximum(agg * dis[:, None] + b_ref[...], 0.0)


def _tc_gru_body(e_ref, h_ref, wir, wiz, win, whr, whz, whn,
                 bir, biz, bin_, bhr, bhz, bhn, o_ref):
    def step(t, h):
        xt = e_ref[t]
        mm = lambda a, w: jnp.dot(a, w[...], preferred_element_type=jnp.float32)
        r = jax.nn.sigmoid(mm(xt, wir) + bir[...] + mm(h, whr) + bhr[...])
        z = jax.nn.sigmoid(mm(xt, wiz) + biz[...] + mm(h, whz) + bhz[...])
        n = jnp.tanh(mm(xt, win) + bin_[...] + r * (mm(h, whn) + bhn[...]))
        return (1.0 - z) * n + z * h

    o_ref[...] = lax.fori_loop(0, e_ref.shape[0], step, h_ref[...])


def _tc_tables_body(h_ref, t_ref, w1a, w1b, wta, wtb, b1, bt,
                    at1_ref, bt2_ref):
    mm = lambda a, w: jnp.dot(a, w[...], preferred_element_type=jnp.float32)
    a = mm(h_ref[...], w1a) + b1[...]
    b = mm(h_ref[...], w1b)
    t1 = mm(t_ref[...], wta) + bt[...]
    t2 = mm(t_ref[...], wtb)
    at1_ref[...] = jnp.concatenate([a, t1], axis=1)
    bt2_ref[...] = jnp.concatenate([b, t2], axis=1)


RFIN = 256


def _tc_final_body(d_ref, bd_ref, o0_ref, o1_ref):
    delta = d_ref[...] + bd_ref[0, 0]
    def nsp(x):
        return -(jnp.maximum(x, 0.0) + jnp.log1p(jnp.exp(-jnp.abs(x))))
    o0_ref[...] = nsp(-delta)
    o1_ref[...] = nsp(delta)


def _deg_spec():
    return pl.BlockSpec((NC, R, 1), lambda i: (0, i, 0))


def _full(shape):
    return pl.BlockSpec(shape, lambda i: tuple(0 for _ in shape))


_g1_call = pl.pallas_call(
    _tc_g1_body,
    grid=(N // R,),
    in_specs=[pl.BlockSpec((R, D), lambda i: (i, 0)),
              _full((D, H)),
              _deg_spec()],
    out_specs=pl.BlockSpec((R, H), lambda i: (i, 0)),
    out_shape=jax.ShapeDtypeStruct((N, H), jnp.float32),
)

_g2_call = pl.pallas_call(
    _tc_g2_body,
    grid=(N // R,),
    in_specs=[pl.BlockSpec((NC, R, H), lambda i: (0, i, 0)),
              _deg_spec(),
              _full((1, H)),
              _full((H, H))],
    out_specs=pl.BlockSpec((R, H), lambda i: (i, 0)),
    out_shape=jax.ShapeDtypeStruct((N, H), jnp.float32),
)

_h2_call = pl.pallas_call(
    _tc_h2_body,
    grid=(N // R,),
    in_specs=[pl.BlockSpec((NC, R, H), lambda i: (0, i, 0)),
              _deg_spec(),
              _full((1, H))],
    out_specs=pl.BlockSpec((R, H), lambda i: (i, 0)),
    out_shape=jax.ShapeDtypeStruct((N, H), jnp.float32),
)

def _make_gru_call(nsteps):
    return pl.pallas_call(
        _tc_gru_body,
        grid=(N // R,),
        in_specs=[pl.BlockSpec((nsteps, R, TF), lambda i: (0, i, 0)),
                  pl.BlockSpec((R, H), lambda i: (i, 0))]
        + [_full((TF, H))] * 3 + [_full((H, H))] * 3 + [_full((1, H))] * 6,
        out_specs=pl.BlockSpec((R, H), lambda i: (i, 0)),
        out_shape=jax.ShapeDtypeStruct((N, H), jnp.float32),
    )


_gru_call = _make_gru_call(L)
_gru_half_call = _make_gru_call(L // 2)

_tables_call = pl.pallas_call(
    _tc_tables_body,
    grid=(N // R,),
    in_specs=[pl.BlockSpec((R, H), lambda i: (i, 0)),
              pl.BlockSpec((R, H), lambda i: (i, 0))]
    + [_full((H, H))] * 4 + [_full((1, H))] * 2,
    out_specs=[pl.BlockSpec((R, 2 * H), lambda i: (i, 0)),
               pl.BlockSpec((R, 2 * H), lambda i: (i, 0))],
    out_shape=[jax.ShapeDtypeStruct((N, 2 * H), jnp.float32),
               jax.ShapeDtypeStruct((N, 2 * H), jnp.float32)],
)


def _make_final_call(E):
    assert E == NCHUNK_RO * CHR and NCHUNK_RO % RFIN == 0
    return pl.pallas_call(
        _tc_final_body,
        grid=(NCHUNK_RO // RFIN,),
        in_specs=[pl.BlockSpec((RFIN, CHR), lambda i: (i, 0)),
                  _full((1, 1))],
        out_specs=[pl.BlockSpec((RFIN, CHR), lambda i: (i, 0)),
                   pl.BlockSpec((RFIN, CHR), lambda i: (i, 0))],
        out_shape=[jax.ShapeDtypeStruct((NCHUNK_RO, CHR), jnp.float32),
                   jax.ShapeDtypeStruct((NCHUNK_RO, CHR), jnp.float32)],
    )


def kernel(x, edge_index, xtext, conv1_W, conv1_b, conv2_W, conv2_b, embed,
           W_ih, W_hh, b_ih, b_hh, lin1_W, lin1_b, lint_W, lint_b,
           linf_W, linf_b):
    E = edge_index.shape[1]
    src0 = edge_index[0]
    dst0 = edge_index[1]
    loop = jnp.asarray(np.arange(N, dtype=np.int32))

    npad_c = EP_CONV - (E + N)
    pad_src = jnp.asarray(np.arange(npad_c, dtype=np.int32) % N)
    pad_dst = jnp.asarray(
        N + np.arange(npad_c, dtype=np.int32) % (NPAD - N))
    src3c = jnp.concatenate([src0, loop, pad_src]).reshape(NW, J_CONV, CH)
    dst3c = jnp.concatenate([dst0, loop, pad_dst]).reshape(NW, J_CONV, CH)

    fill = jnp.asarray(
        (np.arange(NCHUNK_RO * (CH - CHR), dtype=np.int32)
         % N).reshape(NCHUNK_RO, CH - CHR))
    src3r = jnp.concatenate(
        [src0.reshape(NCHUNK_RO, CHR), fill], axis=1).reshape(NW, J_RO, CH)
    dst3r = jnp.concatenate(
        [dst0.reshape(NCHUNK_RO, CHR), fill], axis=1).reshape(NW, J_RO, CH)

    xt3 = jnp.transpose(xtext).reshape(NW, J_EMB, CHE)

    deg2, emb_tm = _sc_deg_embed(dst3c, xt3, embed)
    deg3 = deg2.reshape(NC, NPAD, 1)

    emb_a = emb_tm[:N * L // 2].reshape(L // 2, N, TF)
    emb_b = emb_tm[N * L // 2:].reshape(L // 2, N, TF)
    wir, wiz, win = (W_ih[:H].T, W_ih[H:2 * H].T, W_ih[2 * H:].T)
    whr, whz, whn = (W_hh[:H].T, W_hh[H:2 * H].T, W_hh[2 * H:].T)
    bir, biz, bin_ = (b_ih[:H].reshape(1, H), b_ih[H:2 * H].reshape(1, H),
                      b_ih[2 * H:].reshape(1, H))
    bhr, bhz, bhn = (b_hh[:H].reshape(1, H), b_hh[H:2 * H].reshape(1, H),
                     b_hh[2 * H:].reshape(1, H))
    gru_w = (wir, wiz, win, whr, whz, whn, bir, biz, bin_, bhr, bhz, bhn)
    h0 = jnp.zeros((N, H), jnp.float32)

    g1 = _g1_call(x, conv1_W, deg3)
    p1 = _sc_conv(src3c, dst3c, g1)
    t10 = _gru_half_call(emb_a, h0, *gru_w)
    p1b, t10b = lax.optimization_barrier((p1, t10))
    g2 = _g2_call(p1b, deg3, conv1_b.reshape(1, H), conv2_W)
    p2 = _sc_conv(src3c, dst3c, g2)
    t = _gru_half_call(emb_b, t10b, *gru_w)
    h2 = _h2_call(p2, deg3, conv2_b.reshape(1, H))

    at1, bt2 = _tables_call(
        h2, t, lin1_W[:H], lin1_W[H:], lint_W[:H], lint_W[H:],
        lin1_b.reshape(1, H), lint_b.reshape(1, H))

    wcat = linf_W[:, 0] - linf_W[:, 1]
    delta = _sc_readout(src3r, dst3r, at1, bt2, wcat)

    bd = (linf_b[0] - linf_b[1]).reshape(1, 1)
    o0, o1 = _make_final_call(E)(delta, bd)
    return jnp.stack([o0.reshape(E), o1.reshape(E)], axis=1)


# speedup vs baseline: 1.3829x; 1.1874x over previous
import functools

import jax
import jax.numpy as jnp
import numpy as np
from jax import lax
from jax.experimental import pallas as pl
from jax.experimental.pallas import tpu as pltpu
from jax.experimental.pallas import tpu_sc as plsc

N = 10000
D = 128
H = 64
TF = 64
L = 20
V = 100000
C = 2

NC = 2
NS = 16
NW = NC * NS
CH = 128

NPAD = 10240
ROWS_PER_SUB = NPAD // NS

J_CONV = 82
EP_CONV = NW * J_CONV * CH
CHR = 125
J_RO = 80
NCHUNK_RO = NW * J_RO
CHE = 125
J_EMB = 50
EP_EMB = NW * J_EMB * CHE
assert EP_EMB == N * L


def _mesh():
    return plsc.VectorSubcoreMesh(
        core_axis_name="c", subcore_axis_name="s",
        num_cores=NC, num_subcores=NS)


@functools.cache
def _build_sc_deg_embed():
    return functools.partial(
        pl.kernel,
        out_type=[
            jax.ShapeDtypeStruct((NC, NPAD), jnp.float32),
            jax.ShapeDtypeStruct((EP_EMB // 2, 2 * TF), jnp.float32),
        ],
        mesh=_mesh(),
        scratch_types=[
            pltpu.VMEM((J_CONV, CH), jnp.int32),
            pltpu.VMEM((J_EMB, CHE), jnp.int32),
            pltpu.VMEM((CH,), jnp.float32),
            pltpu.VMEM((CHE, TF), jnp.float32),
            pltpu.VMEM((CHE, TF), jnp.float32),
            pltpu.VMEM((ROWS_PER_SUB,), jnp.float32),
            pltpu.VMEM_SHARED((NPAD,), jnp.float32),
            pltpu.SemaphoreType.DMA,
            pltpu.SemaphoreType.DMA,
            pltpu.SemaphoreType.DMA,
            pltpu.SemaphoreType.DMA,
        ],
        compiler_params=pltpu.CompilerParams(use_tc_tiling_on_sc=False),
    )(_sc_deg_embed_body)


def _sc_deg_embed(dst3, xt3, embed):
    return _build_sc_deg_embed()(dst3, xt3, embed)


def _sc_deg_embed_body(dst3_hbm, xt3_hbm, embed_hbm, deg_out, emb_out,
                       didx, eidx, ones_v, gb0, gb1, bnc, deg_sh,
                       es0, es1, ws0, ws1):
    cid = lax.axis_index("c")
    sid = lax.axis_index("s")
    wid = sid * NC + cid

    def zb(i, carry):
        bnc[pl.ds(i * 16, 16)] = jnp.zeros((16,), jnp.float32)
        return carry
    lax.fori_loop(0, ROWS_PER_SUB // 16, zb, 0)
    pltpu.sync_copy(bnc, deg_sh.at[pl.ds(sid * ROWS_PER_SUB, ROWS_PER_SUB)])

    def ob(i, carry):
        ones_v[pl.ds(i * 16, 16)] = jnp.ones((16,), jnp.float32)
        return carry
    lax.fori_loop(0, CH // 16, ob, 0)
    plsc.subcore_barrier()

    pltpu.sync_copy(dst3_hbm.at[wid], didx)
    pltpu.sync_copy(xt3_hbm.at[wid], eidx)

    cpt = N // CHE

    def _emb_dst(j):
        c = wid * J_EMB + j
        t = c // cpt
        r0 = (t // 2) * N + (c % cpt) * CHE
        cofs = (t % 2) * TF
        return emb_out.at[pl.ds(r0, CHE), pl.ds(cofs, TF)]

    pltpu.async_copy(embed_hbm.at[eidx.at[0]], gb0, es0)
    pltpu.async_copy(embed_hbm.at[eidx.at[1]], gb1, es1)

    def ebody(j2, carry):
        for p, gb, es, ws in ((0, gb0, es0, ws0), (1, gb1, es1, ws1)):
            j = 2 * j2 + p
            pltpu.make_async_copy(embed_hbm.at[eidx.at[j]], gb, es).wait()
            pltpu.async_copy(gb, _emb_dst(j), ws)
            pltpu.make_async_copy(gb, _emb_dst(j), ws).wait()
            nj = jnp.minimum(j + 2, J_EMB - 1)
            pltpu.async_copy(embed_hbm.at[eidx.at[nj]], gb, es)
        return carry
    lax.fori_loop(0, J_EMB // 2, ebody, 0)
    pltpu.make_async_copy(embed_hbm.at[eidx.at[J_EMB - 1]], gb0, es0).wait()
    pltpu.make_async_copy(embed_hbm.at[eidx.at[J_EMB - 1]], gb1, es1).wait()

    def body(j, carry):
        pltpu.sync_copy(ones_v, deg_sh.at[didx.at[j]], add=True)
        return carry
    lax.fori_loop(0, J_CONV, body, 0)
    plsc.subcore_barrier()

    pltpu.sync_copy(deg_sh.at[pl.ds(sid * ROWS_PER_SUB, ROWS_PER_SUB)], bnc)
    pltpu.sync_copy(bnc, deg_out.at[cid, pl.ds(sid * ROWS_PER_SUB, ROWS_PER_SUB)])


@functools.cache
def _build_sc_conv():
    return functools.partial(
        pl.kernel,
        out_type=jax.ShapeDtypeStruct((NC, NPAD, H), jnp.float32),
        mesh=_mesh(),
        scratch_types=[
            pltpu.VMEM((J_CONV, CH), jnp.int32),
            pltpu.VMEM((J_CONV, CH), jnp.int32),
            pltpu.VMEM((CH, H), jnp.float32),
            pltpu.VMEM((CH, H), jnp.float32),
            pltpu.VMEM((ROWS_PER_SUB, H), jnp.float32),
            pltpu.VMEM_SHARED((NPAD, H), jnp.float32),
            pltpu.SemaphoreType.DMA,
            pltpu.SemaphoreType.DMA,
        ],
        compiler_params=pltpu.CompilerParams(use_tc_tiling_on_sc=False),
    )(_sc_conv_body)


def _sc_conv(src3, dst3, g):
    return _build_sc_conv()(src3, dst3, g)


def _sc_conv_body(src3_hbm, dst3_hbm, g_hbm, part_out,
                  sidx, didx, buf0, buf1, zbuf, agg_sh, sem0, sem1):
    cid = lax.axis_index("c")
    sid = lax.axis_index("s")
    wid = sid * NC + cid

    def zb(i, carry):
        r = i // 4
        k = i % 4
        zbuf[r, pl.ds(k * 16, 16)] = jnp.zeros((16,), jnp.float32)
        return carry
    lax.fori_loop(0, ROWS_PER_SUB * 4, zb, 0)
    pltpu.sync_copy(zbuf, agg_sh.at[pl.ds(sid * ROWS_PER_SUB, ROWS_PER_SUB)])
    plsc.subcore_barrier()

    pltpu.sync_copy(src3_hbm.at[wid], sidx)
    pltpu.sync_copy(dst3_hbm.at[wid], didx)

    pltpu.async_copy(g_hbm.at[sidx.at[0]], buf0, sem0)
    pltpu.async_copy(g_hbm.at[sidx.at[1]], buf1, sem1)

    def body(j2, carry):
        for p, buf, sem in ((0, buf0, sem0), (1, buf1, sem1)):
            j = 2 * j2 + p
            pltpu.make_async_copy(g_hbm.at[sidx.at[j]], buf, sem).wait()
            pltpu.sync_copy(buf, agg_sh.at[didx.at[j]], add=True)
            nj = jnp.minimum(j + 2, J_CONV - 1)
            pltpu.async_copy(g_hbm.at[sidx.at[nj]], buf, sem)
        return carry
    lax.fori_loop(0, J_CONV // 2, body, 0)
    pltpu.make_async_copy(g_hbm.at[sidx.at[J_CONV - 1]], buf0, sem0).wait()
    pltpu.make_async_copy(g_hbm.at[sidx.at[J_CONV - 1]], buf1, sem1).wait()
    plsc.subcore_barrier()

    pltpu.sync_copy(agg_sh.at[pl.ds(sid * ROWS_PER_SUB, ROWS_PER_SUB)], zbuf)
    pltpu.sync_copy(zbuf, part_out.at[cid, pl.ds(sid * ROWS_PER_SUB, ROWS_PER_SUB)])


@functools.cache
def _build_sc_readout():
    return functools.partial(
        pl.kernel,
        out_type=jax.ShapeDtypeStruct((NCHUNK_RO, CHR), jnp.float32),
        mesh=_mesh(),
        scratch_types=[
            pltpu.VMEM((J_RO, CH), jnp.int32),
            pltpu.VMEM((J_RO, CH), jnp.int32),
            pltpu.VMEM((CH, 2 * H), jnp.float32),
            pltpu.VMEM((CH, 2 * H), jnp.float32),
            pltpu.VMEM((CH, 2 * H), jnp.float32),
            pltpu.VMEM((CH, 2 * H), jnp.float32),
            pltpu.VMEM((CH,), jnp.float32),
            pltpu.VMEM((CH,), jnp.float32),
            pltpu.VMEM((2 * H,), jnp.float32),
            pltpu.SemaphoreType.DMA,
            pltpu.SemaphoreType.DMA,
            pltpu.SemaphoreType.DMA,
            pltpu.SemaphoreType.DMA,
        ],
        compiler_params=pltpu.CompilerParams(
            use_tc_tiling_on_sc=False, needs_layout_passes=False),
    )(_sc_readout_body)


def _sc_readout(src3, dst3, at1, bt2, wcat):
    return _build_sc_readout()(src3, dst3, at1, bt2, wcat)


def _sc_readout_body(src3_hbm, dst3_hbm, at1_hbm, bt2_hbm, wcat_hbm,
                     delta_out, sidx, didx, ba0, bb0, ba1, bb1, db0, db1,
                     wv, gs0, gs1, ws0, ws1):
    cid = lax.axis_index("c")
    sid = lax.axis_index("s")
    wid = sid * NC + cid
    base_row = wid * J_RO

    pltpu.sync_copy(src3_hbm.at[wid], sidx)
    pltpu.sync_copy(dst3_hbm.at[wid], didx)
    pltpu.sync_copy(wcat_hbm, wv)
    wregs = [wv[pl.ds(k * 16, 16)] for k in range(2 * H // 16)]
    lane = lax.iota(jnp.int32, 16)
    masks = [lane == e for e in range(16)]
    shuf = [lane ^ (1 << b) for b in range(4)]

    dnums = lax.GatherDimensionNumbers(
        offset_dims=(), collapsed_slice_dims=(0,), start_index_map=(0,))

    def _lanesum(v):
        for s in shuf:
            v = v + lax.gather(
                v, s[:, None], dimension_numbers=dnums, slice_sizes=(1,),
                mode=lax.GatherScatterMode.PROMISE_IN_BOUNDS)
        return v

    pltpu.async_copy(at1_hbm.at[sidx.at[0]], ba0, gs0)
    pltpu.async_copy(bt2_hbm.at[didx.at[0]], bb0, gs0)
    pltpu.async_copy(at1_hbm.at[sidx.at[1]], ba1, gs1)
    pltpu.async_copy(bt2_hbm.at[didx.at[1]], bb1, gs1)

    def body(j2, carry):
        for p, ba, bb, db, gs, ws in (
                (0, ba0, bb0, db0, gs0, ws0), (1, ba1, bb1, db1, gs1, ws1)):
            j = 2 * j2 + p
            pltpu.make_async_copy(at1_hbm.at[sidx.at[j]], ba, gs).wait()
            pltpu.make_async_copy(bt2_hbm.at[didx.at[j]], bb, gs).wait()

            @pl.when(j2 >= 1)
            def _():
                pltpu.make_async_copy(
                    db.at[pl.ds(0, CHR)], delta_out.at[base_row], ws).wait()

            def edge_grp(g, c2):
                acc16 = jnp.zeros((16,), jnp.float32)
                for e in range(16):
                    i = g * 16 + e
                    acc = jnp.maximum(
                        ba[i, pl.ds(0, 16)] + bb[i, pl.ds(0, 16)],
                        0.0) * wregs[0]
                    for k in range(1, 2 * H // 16):
                        acc = acc + jnp.maximum(
                            ba[i, pl.ds(k * 16, 16)]
                            + bb[i, pl.ds(k * 16, 16)], 0.0) * wregs[k]
                    acc16 = jnp.where(masks[e], _lanesum(acc), acc16)
                db[pl.ds(g * 16, 16)] = acc16
                return c2
            lax.fori_loop(0, CH // 16, edge_grp, 0)

            nj = jnp.minimum(j + 2, J_RO - 1)
            pltpu.async_copy(at1_hbm.at[sidx.at[nj]], ba, gs)
            pltpu.async_copy(bt2_hbm.at[didx.at[nj]], bb, gs)
            pltpu.async_copy(
                db.at[pl.ds(0, CHR)], delta_out.at[base_row + j], ws)
        return carry
    lax.fori_loop(0, J_RO // 2, body, 0)

    for ba, bb, db, gs, ws in (
            (ba0, bb0, db0, gs0, ws0), (ba1, bb1, db1, gs1, ws1)):
        pltpu.make_async_copy(at1_hbm.at[sidx.at[J_RO - 1]], ba, gs).wait()
        pltpu.make_async_copy(bt2_hbm.at[didx.at[J_RO - 1]], bb, gs).wait()
        pltpu.make_async_copy(
            db.at[pl.ds(0, CHR)], delta_out.at[base_row], ws).wait()


R = 1000


def _dis(deg_ref):
    deg = deg_ref[0, :, 0] + deg_ref[1, :, 0]
    return jnp.where(deg > 0, 1.0 / jnp.sqrt(deg), 0.0)


def _tc_g1_body(x_ref, w_ref, deg_ref, o_ref):
    dis = _dis(deg_ref)
    xw = jnp.dot(x_ref[...], w_ref[...], preferred_element_type=jnp.float32)
    o_ref[...] = xw * dis[:, None]


def _tc_g2_body(p_ref, deg_ref, b_ref, w_ref, o_ref):
    dis = _dis(deg_ref)
    agg = p_ref[0] + p_ref[1]
    h1 = jnp.maximum(agg * dis[:, None] + b_ref[...], 0.0)
    o_ref[...] = jnp.dot(
        h1, w_ref[...], preferred_element_type=jnp.float32) * dis[:, None]


def _tc_h2_body(p_ref, deg_ref, b_ref, o_ref):
    dis = _dis(deg_ref)
    agg = p_ref[0] + p_ref[1]
    o_ref[...] = jnp.maximum(agg * dis[:, None] + b_ref[...], 0.0)


def _tc_gru_body(e_ref, h_ref, wir_e, wir_o, wiz_e, wiz_o, win_e, win_o,
                 whr, whz, whn, bir, biz, bin_, bhr, bhz, bhn, o_ref):
    mm = lambda a, w: jnp.dot(a, w[...], preferred_element_type=jnp.float32)

    def step(t2, h):
        x2 = e_ref[t2]
        for we_r, we_z, we_n in ((wir_e, wiz_e, win_e),
                                 (wir_o, wiz_o, win_o)):
            r = jax.nn.sigmoid(mm(x2, we_r) + bir[...] + mm(h, whr) + bhr[...])
            z = jax.nn.sigmoid(mm(x2, we_z) + biz[...] + mm(h, whz) + bhz[...])
            n = jnp.tanh(mm(x2, we_n) + bin_[...] + r * (mm(h, whn) + bhn[...]))
            h = (1.0 - z) * n + z * h
        return h

    o_ref[...] = lax.fori_loop(0, e_ref.shape[0], step, h_ref[...])


def _tc_tables_body(h_ref, t_ref, w1a, w1b, wta, wtb, b1, bt,
                    at1_ref, bt2_ref):
    mm = lambda a, w: jnp.dot(a, w[...], preferred_element_type=jnp.float32)
    a = mm(h_ref[...], w1a) + b1[...]
    b = mm(h_ref[...], w1b)
    t1 = mm(t_ref[...], wta) + bt[...]
    t2 = mm(t_ref[...], wtb)
    at1_ref[...] = jnp.concatenate([a, t1], axis=1)
    bt2_ref[...] = jnp.concatenate([b, t2], axis=1)


RFIN = 256


def _tc_final_body(d_ref, bd_ref, o0_ref, o1_ref):
    delta = d_ref[...] + bd_ref[0, 0]
    def nsp(x):
        return -(jnp.maximum(x, 0.0) + jnp.log1p(jnp.exp(-jnp.abs(x))))
    o0_ref[...] = nsp(-delta)
    o1_ref[...] = nsp(delta)


def _deg_spec():
    return pl.BlockSpec((NC, R, 1), lambda i: (0, i, 0))


def _full(shape):
    return pl.BlockSpec(shape, lambda i: tuple(0 for _ in shape))


_g1_call = pl.pallas_call(
    _tc_g1_body,
    grid=(N // R,),
    in_specs=[pl.BlockSpec((R, D), lambda i: (i, 0)),
              _full((D, H)),
              _deg_spec()],
    out_specs=pl.BlockSpec((R, H), lambda i: (i, 0)),
    out_shape=jax.ShapeDtypeStruct((N, H), jnp.float32),
)

_g2_call = pl.pallas_call(
    _tc_g2_body,
    grid=(N // R,),
    in_specs=[pl.BlockSpec((NC, R, H), lambda i: (0, i, 0)),
              _deg_spec(),
              _full((1, H)),
              _full((H, H))],
    out_specs=pl.BlockSpec((R, H), lambda i: (i, 0)),
    out_shape=jax.ShapeDtypeStruct((N, H), jnp.float32),
)

_h2_call = pl.pallas_call(
    _tc_h2_body,
    grid=(N // R,),
    in_specs=[pl.BlockSpec((NC, R, H), lambda i: (0, i, 0)),
              _deg_spec(),
              _full((1, H))],
    out_specs=pl.BlockSpec((R, H), lambda i: (i, 0)),
    out_shape=jax.ShapeDtypeStruct((N, H), jnp.float32),
)

def _make_gru_call(npairs):
    return pl.pallas_call(
        _tc_gru_body,
        grid=(N // R,),
        in_specs=[pl.BlockSpec((npairs, R, 2 * TF), lambda i: (0, i, 0)),
                  pl.BlockSpec((R, H), lambda i: (i, 0))]
        + [_full((2 * TF, H))] * 6 + [_full((H, H))] * 3
        + [_full((1, H))] * 6,
        out_specs=pl.BlockSpec((R, H), lambda i: (i, 0)),
        out_shape=jax.ShapeDtypeStruct((N, H), jnp.float32),
    )


_gru_half_call = _make_gru_call(L // 4)

_tables_call = pl.pallas_call(
    _tc_tables_body,
    grid=(N // R,),
    in_specs=[pl.BlockSpec((R, H), lambda i: (i, 0)),
              pl.BlockSpec((R, H), lambda i: (i, 0))]
    + [_full((H, H))] * 4 + [_full((1, H))] * 2,
    out_specs=[pl.BlockSpec((R, 2 * H), lambda i: (i, 0)),
               pl.BlockSpec((R, 2 * H), lambda i: (i, 0))],
    out_shape=[jax.ShapeDtypeStruct((N, 2 * H), jnp.float32),
               jax.ShapeDtypeStruct((N, 2 * H), jnp.float32)],
)


def _make_final_call(E):
    assert E == NCHUNK_RO * CHR and NCHUNK_RO % RFIN == 0
    return pl.pallas_call(
        _tc_final_body,
        grid=(NCHUNK_RO // RFIN,),
        in_specs=[pl.BlockSpec((RFIN, CHR), lambda i: (i, 0)),
                  _full((1, 1))],
        out_specs=[pl.BlockSpec((RFIN, CHR), lambda i: (i, 0)),
                   pl.BlockSpec((RFIN, CHR), lambda i: (i, 0))],
        out_shape=[jax.ShapeDtypeStruct((NCHUNK_RO, CHR), jnp.float32),
                   jax.ShapeDtypeStruct((NCHUNK_RO, CHR), jnp.float32)],
    )


def kernel(x, edge_index, xtext, conv1_W, conv1_b, conv2_W, conv2_b, embed,
           W_ih, W_hh, b_ih, b_hh, lin1_W, lin1_b, lint_W, lint_b,
           linf_W, linf_b):
    E = edge_index.shape[1]
    src0 = edge_index[0]
    dst0 = edge_index[1]
    loop = jnp.asarray(np.arange(N, dtype=np.int32))

    npad_c = EP_CONV - (E + N)
    pad_src = jnp.asarray(np.arange(npad_c, dtype=np.int32) % N)
    pad_dst = jnp.asarray(
        N + np.arange(npad_c, dtype=np.int32) % (NPAD - N))
    src3c = jnp.concatenate([src0, loop, pad_src]).reshape(NW, J_CONV, CH)
    dst3c = jnp.concatenate([dst0, loop, pad_dst]).reshape(NW, J_CONV, CH)

    fill = jnp.asarray(
        (np.arange(NCHUNK_RO * (CH - CHR), dtype=np.int32)
         % N).reshape(NCHUNK_RO, CH - CHR))
    src3r = jnp.concatenate(
        [src0.reshape(NCHUNK_RO, CHR), fill], axis=1).reshape(NW, J_RO, CH)
    dst3r = jnp.concatenate(
        [dst0.reshape(NCHUNK_RO, CHR), fill], axis=1).reshape(NW, J_RO, CH)

    xt3 = jnp.transpose(xtext).reshape(NW, J_EMB, CHE)

    deg2, emb_tm = _sc_deg_embed(dst3c, xt3, embed)
    deg3 = deg2.reshape(NC, NPAD, 1)

    emb_a = emb_tm[:N * L // 4].reshape(L // 4, N, 2 * TF)
    emb_b = emb_tm[N * L // 4:].reshape(L // 4, N, 2 * TF)
    wir, wiz, win = (W_ih[:H].T, W_ih[H:2 * H].T, W_ih[2 * H:].T)
    whr, whz, whn = (W_hh[:H].T, W_hh[H:2 * H].T, W_hh[2 * H:].T)
    zpad = jnp.zeros((TF, H), jnp.float32)
    we = lambda w: jnp.concatenate([w, zpad], axis=0)
    wo = lambda w: jnp.concatenate([zpad, w], axis=0)
    bir, biz, bin_ = (b_ih[:H].reshape(1, H), b_ih[H:2 * H].reshape(1, H),
                      b_ih[2 * H:].reshape(1, H))
    bhr, bhz, bhn = (b_hh[:H].reshape(1, H), b_hh[H:2 * H].reshape(1, H),
                     b_hh[2 * H:].reshape(1, H))
    gru_w = (we(wir), wo(wir), we(wiz), wo(wiz), we(win), wo(win),
             whr, whz, whn, bir, biz, bin_, bhr, bhz, bhn)
    h0 = jnp.zeros((N, H), jnp.float32)

    g1 = _g1_call(x, conv1_W, deg3)
    p1 = _sc_conv(src3c, dst3c, g1)
    t10 = _gru_half_call(emb_a, h0, *gru_w)
    p1b, t10b = lax.optimization_barrier((p1, t10))
    g2 = _g2_call(p1b, deg3, conv1_b.reshape(1, H), conv2_W)
    p2 = _sc_conv(src3c, dst3c, g2)
    t = _gru_half_call(emb_b, t10b, *gru_w)
    h2 = _h2_call(p2, deg3, conv2_b.reshape(1, H))

    at1, bt2 = _tables_call(
        h2, t, lin1_W[:H], lin1_W[H:], lint_W[:H], lint_W[H:],
        lin1_b.reshape(1, H), lint_b.reshape(1, H))

    wcat = linf_W[:, 0] - linf_W[:, 1]
    delta = _sc_readout(src3r, dst3r, at1, bt2, wcat)

    bd = (linf_b[0] - linf_b[1]).reshape(1, 1)
    o0, o1 = _make_final_call(E)(delta, bd)
    return jnp.stack([o0.reshape(E), o1.reshape(E)], axis=1)
